# Initial kernel scaffold; baseline (speedup 1.0000x reference)
import jax
import jax.numpy as jnp
from jax.experimental import pallas as pl


def kernel(x):
    raise NotImplementedError("write your pallas kernel here")


# speedup vs baseline: 2.6050x; 2.6050x over previous
import base64
import functools
import zlib

import jax
import jax.numpy as jnp
import numpy as np
from jax import lax
from jax.experimental import pallas as pl
from jax.experimental.pallas import tpu as pltpu
from jax.experimental.pallas import tpu_sc as plsc

_B = 4
_N = 4096
_D = 2048
_ROWS = _B * _N

_PERM_B85 = (
    "c-jrr0|FR2768E5d8uvNwr$(C-TvCPZQHhO+qSzqhnU_WqhH`p@@V0^sR$*QVL#j#QW7Sp{sDMLlg8brmslg6JB$s<L_Y7(Ri9Qp"
    "!dp-(RWN)ExBLZLDO}ON@jEd|rxm4bS-%{g=E=iJR?2jjxA;VtS0|QhY&6^mr<2vm5!|1y)vM5+P=_Xj2yj~thoPuE+$VGCaXQV9"
    "bxZJOcF_s(6jHcZZiG%3s`)W?6#J_R%51)dUrx@d(lmkUBCf$AwIY1R?ZiE`*xtfQ<x^Qud$T}p@omfux`U12F@(e`)lOW5PFBlo"
    "3*X465SeXT@zzZ;CBrAZmW~Qr`3-VP4G7Ix%Fqyx6J2R>`qH)+XKjC6Q%O`4uaSLea(_%FgYx!>8zobto&1}>1iyF(yBAVG2Kxq="
    "6xGaajKe9{-R;8T^)lK~&2usA0Q5x0Mft@nvcM%F#I40|MOFOUrF8dMT6sa`f@yG0jKEWA9Tf>bhq0oPULy0+@;;wvKsTzE;Wsa?"
    "YM4#phdSz7Lp&HG#_}a7l1^$&7^`-nvveCdh%=Gj_KZhzy_WXB5Y;8OCCDJu9h!jGuV_@-S6@Uo%xGBx4$74J7!Z_`a+i<fHI-xs"
    "yq7fNFa2!&#2#a5P#IoS*EeO{c=1R?Q^jmjkq5=4kwqWAk9>kdP}an;(eXOD-n<}FO%~TuoiGkdw@VC_J^cc;&F?k6X^&7tyd;-d"
    "H1jOX$A?TNIU0{Jk69ewQd;`Sr4Q5}rGVp!(V+_(K&t@I*?1Efz=o6Vs2<OU44fvZF%3P)KA4B+*>kEtdy6}vkEpHN4r$0HU5RJm"
    "^L;&4kWUd)<j*kL{laTqV!njd_1DR8Q&T47`|Jla1~nH~d~UYR^fhNlOgLlT%LaNGxrjUQO01hDYOy@68rT)2uDOSo@>M*Me-bLI"
    "&U%1&gxW%PmIzvisO+B2t_a;mqr1!QzG@)m@$Gax8Utf}XE#{}eHOPgQS@bg6-uBkG7U}|UbDB1qY<LAh_9rIMzc!}F>wo4$|hx%"
    "T?IFs@0TsY16D)bU~5GZH;J~j75re8$|Czi7eWPCIh~yCz)M2@kXYqbR2DOb(G${~r{{mdw-ALVhJo$~AI^4?^(ZcRV~IJ1U$U*T"
    "B+jOD%M-LBzN-q6t#DYkW!J=VQ$`=4?`<pnDtx64{AN(Lb$CsG@)e>z>*pV<#?XzWwU2mA@zlnX!*LquDAVJ?cB#mwy7^3WyZOYn"
    "vJ2sw-_99og_y6xQ~6Y;5{xqu@LFSSX)+d9qql7Yx5QkRMMFK;i6s`(LkiMbE@6d3Trp3!rNz-<n_N~5O(74BA%~dM_Bww7M^Fm6"
    "TJ=L8<XgRgJ=1yY5VQ&p@LS1dUD+QBQ_L^5TkNz~NNL_qHn9uPOE~CaxWaTiFKpV-SpJ<(W=e|+D2}^_p6g<C932@BhAoiE3bIP3"
    "B$UNfd*l?i)Xt;H=`QnA#K8|`DcnkS(}i)N@Lsp^18G9n%Uo24vC^gVP0}x1foE`vUkX{o0eM%K@x%08lgQk0NpUwgCCZDAHXVD<"
    "ULYsehL~oYk4YQaeljQd$+RpV&fplZ%+=LRLQAqoU9??g8XI5Nmq%?TwcBmx{YgJlBs5_gV1kRJKH9ptJf~1aT;SKx6u@qm%4~|@"
    "`=O>E@20Big7b2AT!>A%D9e_jEn&F2&4<x{WKBqb2<y%-nLn_|#bbwkDx4oRV#VD)Uxr>s-&G^|$;OmXaT8KI{3IL1AJLR1W3A~S"
    "u4x5+7rsIbc^K}<{AwL9Mel`<I<tBMMeTL-7fy;!q=|W@uBylKgy^Z>;ZY_pKSz#>p{9w>VA_)V0=et%09hic%2~n7-DIa9i87O|"
    "zM4D1Ht|HH9ItG;t5pR1k|YuBU~(Z&&X@~6nQG~KlA?TxUuRqUg&z6MWSPjIQ<~Yd9_l0e*+;&o{7;O96}qqOM>Fb~XfDfvql%8`"
    "E{;sMh+(`I)bgj~5#1kh@M`L-j8EH;T4)8@!UAvSu7x8CLNb}fC|uBN&_}{Ymro{@J=h;U9Rif3{nZ6rIb=d3Q1ehkRTTY1UOG{1"
    "b|u*_I)oLn9r!(`$Q7Q`E#p^F5s^;yVHv}fFx7Wcb3_$$ja)Ok{3)~#y;D8$cfBBd#Xx@QyKX|5&vWrubXy3lrri))*njL8o{JyJ"
    "qNXgF#(wK9Hb0q&o0<0ZfxRKpqtxNAeGs0jdZwa@NWl#bZP<A|kbSX@#VC4%D%h$fn3epBh>w_@LAs%w{4Vjlhvz1j8(<!i+jK}c"
    "szE6}B0LW{<!<v(T@Bw{53w5FuybgR%P8JiOX7ysY@?ePTDfAfl5a|Gt4VN*J&>i{R5?K>aGm5t6A2chXy}rRFNcw~W{;2J3xij2"
    "fWRHU6h@P~Jhjm*g6#!q<S|uH6%&VD_i#qP<GKBI(woebzu*|1q&{1MhoL$Cl52%u=+-<1yYDKKYxKO2hn}Nrye~h2UnoHrt8TIl"
    "Q2ha4wJXdPlF=QqCEWsT$O>rZU-;}Ov!8B?^FiSw&dO(rO6nK->mQMb;w)-zrjjpou3b!$xss+U-cOr_+~~E)>kimu?43)B_KH5L"
    "HW8sPUvA^-gk*;)1ZB)Dyn}C)j`k9tHP$Zlw^@BP8=Yi=sAvj~V6<o^-=iQ_($%OQP`*vnVll~99Fx~zrNT9L%bizqs0ypuSiRFk"
    "^wj=Ql;)6gSY8$>WT#bR1vyRph8y0p3I3>vW((@AD5l?FR=Gpul^95}&?qpP?L_lMd-~h_Rt-p1xG6{B?Ru*Cs4v-Xc8_N2KCBnb"
    "ttK06W!=PIv!B%?yIqzL`Aip^P<=JM^)?tK`jFuBiY{!1mgFINjLw9PK8bD{KATP9w24m&t9Ics-fW)o4`e8=;`ZzPA)-v}ckwF}"
    ";c>2ztRH5(j3|c3VulGQzBx_TE5gB)rCaG5P2Cg}S*ODr{SG$~&Eloty2=~o!4cn540n#@M1y4mjL{LfBUED@!$VQk&Z0zL7BkQ="
    "yWY)4P85&{WMVdj8pmmcup5t7l_9Eq2$%2y-q$sj7fobe!W2*mjfEb3oUW)U)5u2S9<rY3qg$f&qzP~2>e2N+p6?*e@QnPnuHxI-"
    "Rf4nmXge!PxAC+(A`a?N*a6q{Yt+f7#C54)DRg2SjdkQwFLir(8GA@R!B2dB{E}voX$+R}^eyXccGl75gITN>Pp$LwbG(geE}yy>"
    "thB1nDQc{TspNv|NntMP?EB(ksE2v2>#@G<2#bqn$R4m)MT9bJowy}(+V3Ghz6d2zITsr)(0}5H8EO9M1!N&9%d@elJg@Ac*TWCn"
    "3pG%!d=0#hCPz<rLMi2B7uh!T`DHiW!5?B3a7~?!43y_ZJMvO)P*s#LhiEDJ%oIjv<aauoO_C$jbh%SKCb7gfNN3Bsb2Ku!N{*34"
    "a)xT)ve}ZlpGq&L+FEV_&qBHxi{FuvwyF6ll5mV0isCl9$ELm=<-YqHF1JAlsZ*Z&$v9t#LR({PKFZC$jjRciP;P$Ex8qfiK>zGz"
    "QwOf8uHu7U=Z5*lA-X7sk+>q?!V5Z0*T#?NCy|QR@vm)4_@G*uEh;YWtZJZXWGZ^;2D2A@k9`+@+1z-wedc2Es%)0UEDdjCv)h$="
    "m;8dJi?-+<w1f><=wYmCc&6UtOZ=%!?b7*-@SMK)dqi4P+E+D$(O;Zfq^1j1EIrwDkg-BVmc_?*rNMDb59x1s2kC>O+XCU7j$*3o"
    "TILq*3&lfITUsVCh5c?inf%9h>I*s&{pw?rSZWKcF7J?LDz?5$&%he~$DFriP&QH0XS73EV!KMMm)0MlUF~w3k#vIALPC3(Y2Kn3"
    "c&KmA#<1uzj@hANqYr#;_^SG&^dy`7CC2d@?u)<Y+v5E`BD?R?=#`KeHKDu182!fYve)V5@I>FEv!M#T&qwJ?y0F{BhUz5jpqh-D"
    "$_s9h9wsC5*Zhra$=BM~q%*Fl7qQ9g8&B*f+RijBStF|Gf_|(0M;}2|Cw&)uK<~r7RCQm`{;^A-peVw->I||Ox?xlLoS~IkVA|^;"
    "qz!6}zOtXLCQkq>S$jM)tac+sCtL@=f`;LVN*Lmi@wAiPB(w2yVvKo6|DemHf<75$@QVJGIfs|CSoSH(g@ww+c8V^c8RQbla8y;0"
    ")pkYTs4B#Z`h@xid8Est^nN<;q&k@h`b%hvj>_pOo++Vc^KX*kZmt$S4&MC-gruXf+&)=GuEP~zPAIDrxrKbSJTD)b!)6luBA(;B"
    "vLbrMPKQHono6!J;nOH<NJu+?r*%*sU7lGz+9pADcsbINwZJ1)Lw?z|)kWl4kwMJVNrJ(rU1FDm7B?%+=J3hPq_t!VR2ipKb=)yg"
    "JrtMPEz{Ce6K~}u(TN)0h3C_I?LTwRT({%dJT#Yo=85bg8;kwN-tx(Og>2!XppEPedhZl1E;58U=(XDo_swPXneC;g$OhERJrk8t"
    "0vt(9a-Go;5ydsfyV3dZK?r_GMH5YR2Yph_Cn<bQy3!9v5$M0r1kc2y(Qg0TKWBgGaaN4iwQ)!-J{O9zxUR041+8rbEAS1~CtR^j"
    "d0AC1+=2{rp-qSKu%+ZY%A)hRpZ+7MZL8q4Gy;i(Ci$agEhLcz{ClPSr?AkK#}~tUzL5^J(_~U@^lbCl@1S+qZZ*i4A?0Wn7l~~0"
    "=ODFcz*2=n;v1dj(wmF2u+4{)sQ3`yR;Tr33iLNPKY?Cj9zOXsY=Dz|hP#1#vL9>^sVWBI3?#QLDr(T1-l{}myANg<ZX#-;TbkiY"
    "uDGtCUh*JE+2=MDr1j<9MKQ-F&<YI;Z$sCR-K{n~a0InkeC40iNz{e5CLcmcn}Fujk$DNZk97!v^l(dYc5(-XnN<2^NX;9<ZrhMf"
    "cY__%(cEn)OdiuHwhtWlHAz{UIdlul%zj?O{-%9Jj1W1@RI7Csi%5UIQ(j<Kag7kkjbl?7<2lU-zn`@*t<@ltBXl7j99N^@imFJj"
    "U}R209(mK|B?DwueP3p#Uv*+i)kuGlWj6<PIh7EvqvPxgcbNV5MP1ae*5_k`aT$ET&ksvk19Vq>;&<%}`=9+`_JG#M{7RJ0B;X4}"
    "Y_S(#!|muc)X+~<d(8VV-zA6n@(;epvD)Ao+Z8M?3{^>NZaoLa@TI1g-9xYH#XL31rBCBxc(R|4cCvZ;9o#|_NES93l#e5;^1fuA"
    "KftPy+@zOZjI%*pxd|mFa~%`;&3I_2m(tPvKYT#_04hrBLM)fPsWY=ha-h12Ys>7Okug3Kdxna**RVSLryk<VXudjV^6PZupiSUE"
    "Kw5TKPA5lfS9b#%n(bnUo`E0Ro}w$;6>5;y2HP{V66r5K;U%^P^waf1afl)&vXlBCX^hv(Pb9X#9%AYU{yyH#6Oco=6(4IE{F5C}"
    "+TcS<d(q!^8s#w!!W7?Doinf1JT}*4l<`A9R@-b=8)zZb+7)wk$=+bma0}3ZWW{BmzaA+asY2eW6RMrAU=pE%Xgm7NzM)Y(y(%x`"
    "&@|y5DQ#xyVx%|QL;lL_qK7J>ir59Xq3Wf7`5bJX$SU8Pf9|b~XRDi&c&kYxVyeihv)pKx!x)kfB{2&?xL542?#WNf5_X6@8v+gC"
    "c&HgF^RZ;SeC{@xb)llX8K%)(v<;p}%A0b~8;=P`*-EkkUuE-kM3)n4^Gmd+iG&jAH@q`Dqq4wyyNfjy=WK0%kaZ(U<hO^#E%t&n"
    "Bk#p%e@_09PgO;iheY!u>0id&A2QEmcID(ZH3#R!x6~V&C^)FEtD}wNB7aNo;5xP%S|rBE5wIV9CmPDDxqg6-%O03EdN5rS`iPJ4"
    "UFIMQM5gc=zPK`?0E(eLuyG27@@Oah&OiFHkcyqBt7vDESs%dbAOcS%o|{EvFX~PQ*v#^<8m+h4jiQek82-49GzA~(4$~9R-2dkX"
    "iYPJ(ob}D@AX?eB)VVxkCw(OL0EUTrwjHbrxm0gIhnM0<@pZjhPuFMRvUl!}zJ%KQ33e|(BMR#s;tq=iEy#60mi87Y@MrQ(9Hf8A"
    "A)lDc4$;FR{=t?}>*)@^!={$sZ81nj54eKlrMMA}lbR;2tDy(UOFmD?!$;GqG?{<EABe0h4j<;cIANcQS7;y_tc#Q0va2euuevuz"
    "(n=_c8i!B&QK%wcX!58@Jhn}vKG4W+i>@nYnIfnoG(pQ{J-@`wRuknSh#5Ykk+_a*L?(zurZ|g*$B~K9N@Ne^Of~#Qx5Lr&c3hW_"
    "HI>A7F~@YHEo8Fr{|;$v9-sB358MdXm__F!{7ipbp5QU~C)&ps(`!{Nty}_oQ|)yN*$TTECnbYnw$0&Q;reQ$uFaN))B2JsVsdzn"
    "W3t+68jnhrq3Ak~D+(L<YLbiWS9@G?d{eZfBi(aS-~X0z{Wx@1kCRQq5PT>6VvC`jX~d_x8a%DZN>9N94I(-J$5Xioq5$s<XJu|k"
    "u9K>LhM9j*R2_xYyt!T~U)eu22I?zYqk?WK`=?IYU%H1~rtavLI5L{Ss@RWwB-~~hSv|a2eh#PTS~7wa5;??2brfY4@8|_tmo3%_"
    "MP2mRt@B;UZg!r$L%mfbe+Ol7FHH^=pEpM*X|+&W6JHk=(q(c#PC|=?{P0-3_D6kma}XU1pT!tABMjG5b$*tP7jW<7?U2=$lx1-f"
    "H48n!XI%>_V3mL1--zl^$^5`CL~7Q+<na;YGB|`T(#JS1WKbJnh#h28q8@ZHb2gR<IF9HPnqnJ*jl}D_sAhe@yhPXu#eEy`9nXMu"
    "=r`%@B9aO4l$GQS(HVA~6~|WG*Ns&vHe61pLwGB;92J$%$Szt)KXjFBdOHBG_H)S}-xE@@xjLh&4L?zNQip8ATSZdW4pr4pXf@a-"
    "H`8l!7kQ1_glVX($jKIq=4!5(Pdnl(F1@>DT3I7w&}_J)=;{|jUUkN;GE4OrJ<tzQjp<L@T2IAS=?1^cU4~?~p029elLv^{MwZKn"
    "I)^!q-?#;K5Ff}(sS>`CTEh2}+v01;j{89~T%OLB+d~WW!#5;-RYj83RmYvfKl~qxr&dtKR?`ZsrhAEg(srtSm~1EOg0`)ErT4p@"
    "_AYD8hXCRc(Q&m2zmiGtJsq2EaVPX=H-~ugQQY9SNPN5p&4)GaiCd2csP2A^N@}m;L85e6p_2Oe;tTs7x|n~yAD@Kl3m#5`G8s@K"
    "dMmsLDd}GJmoE?5Tw<LHr4tFlJ64@tHB;<hlLLWYs?MNTvZY%Y2EcQfN;*YkB>6;C<4-k)KD4k{iJw4T9~pPY+g&f3Q)FPzWEz&+"
    "jnqd75x>HB*aa0xe{)M_hu^}`6X-0PW=r5w#F(KX6*=w_v3Mc4#;&GFZoZOLFw1vvMiiu(a8?vY=ck~%lkvC|8bJm@T3GD1;AFfR"
    "okytoE55O_rZX?88nP>90C{ZE%R<zs0%9WnVAk_HHj`^i^5A9RE6PaY;pP6GSmCy*d^{<S#^dpAYz-NNi<0TSKH6`hgvS1+tLs`o"
    "SF=zgceUs&`N+i%bCt*Cd<u)n7nNO~B@xwZ|5>I8B&3uP!akA0j8YyI@sS~kpCs;v!{nuY=6kxsVG7USe}oBmiOv<C;>V#L=?cBr"
    "Q##FLB}H*Hxlm^`fbQ^eEH6$*vV^hnHX0#MgrBkl+aas^eQGe@rkj&><S**$^CE3!NI@HklWw%BqZ`OHbhDkn&*RrJp&g`~`DWrM"
    "87;P$Chn0+4Gmp-)<V~^pHNj&-Hi{^*_4n~rNuvCFj+*W;o3AQEsdVpHhjB1=I-0r<{~*2F2E}L*IlNu(0{a1;5MFH<4=+$<UFfS"
    "W2@D;y%MrjSScDC;eV=`<dtvgKGL43x$DidLIH9GJu!XIc-7n7W1C>Eyovg`IWirzg57c?sYOQ6p1L-Q7=}1!s-jL}zs(Fw{TE(L"
    "4TZAuKqz3xqM4$*?k0-)N#sBBO&{a^Ft9GLRsSbb(VcD-DWb27&L%S37z&v$IFo<nKn|yOeJ+zp?{if`Q+1Pe@s(&@^&Ksgot&|S"
    "WiBb%&X71{GLh^FnTT(ZcT{(~Uvw8Y&|W`;P7M!r0m^JU7DG>_>)0>znbwC#><c)ZUO!~f2%|%3eqWfaK|O<#b5v^ehO9BxY;|Qq"
    "Lpct<kq9UuOXsf2JkXI{hx%kAzM*fjWI8(Si%!|8JT1SdqoYP9q3`1cx&Csth-(JwXwZRH5Gln3*$5)qj^qf;3#Zv)+5ir~1T+jz"
    "=^?%VPRYNSnXCbSYtHii?jPz!hKrbXCQra-_}3vbkD^kjE360VN9vdrKCg{yKbrUSv5&3}i0)=+*rz+&wmheQhBIR;+oLaLx~;Ar"
    ">-Mf}sBVY*m39p7p`+^de2MbrliY((xO3qZdCZU5w`6hHKr-S|BAb!oxXCOExgi3<<nV+YM}`-~lw=YG%sQP%9OY5vNWNT6Q7vt3"
    "d57+m1z<B;#hTMiw3o}kSGtG19E!unvc%@0-DZRx$v(13CXsp%7t9N@$;D9%=y6@0FXxx_JKvSR^(A;I{{vOi<IQ279#%>nPTK`6"
    "H?5_Tp@rx;pXr8@ZLXHMEN7D`WQ4Umfk1kbTBjngRr)b3Hdpx<S<3B)G=43Mja$+4un`qSTSJ{t!gWwL;R5M~FNhX=4K!uH<P@EW"
    "7xMq<F=~^|sp5OEgS|m(@NYCnuNI5&c)LR%)6MjFGT&Ba)50aPLEaJBc@Eu0E%wD&-LNnGG@a~Co)o^jr(qLYh<~~9KC;}%^VzKC"
    "37V(QoAvB2`p;Ei^UVP>i2V?E%u7`(yfy>P1NDr1|1gxG2hbvv1A3!(W<HxCZpb<!8;uT8{Ym+NOvf*94Aa(6)V*zU7zigwbavit"
    "SEt1r6@y$rS4B-0P5xo&^f?ya9SvR45Ve-?(mT+7bj}?V%XNC%8)n)JaMKioA3R&w$D7Efq7f?1w(#X{3_WIwkSN5ur{+AbEn<;b"
    "YOfw@U&xb2^W!$6nF<qFAA8Qf39U#>*Iqs{r_3_XLK0Ddu0>y6HnfH!_7YF@%WxJv%0KiO@DCb~ws8k}em8&@!jr6L^;r?N(R{UW"
    "!thW)KlPo(J=x#)!o9*^aoZ=te@ztplq?tD*-tzZZj)oEZ}>$j%a~{dKL`(KX*`riVbACy|3m&~5~JIummCc_><K@SP8KuldlD6%"
    "^H=<T%q?(v"
)
_PERM = np.frombuffer(
    zlib.decompress(base64.b85decode(_PERM_B85)), dtype="<u2"
).astype(np.int64)
_IDX = (_PERM[None, :] + _N * np.arange(_B)[:, None]).reshape(-1).astype(np.int32)

_NC = 2
_NS = 16
_NW = _NC * _NS
_PER_W = _ROWS // _NW
_CHUNK = 16
_NCHUNK = _PER_W // _CHUNK

@functools.cache
def _build_permute():
    mesh = plsc.VectorSubcoreMesh(core_axis_name="c", subcore_axis_name="s")

    @functools.partial(
        pl.kernel,
        mesh=mesh,
        out_type=jax.ShapeDtypeStruct((_ROWS, _D), jnp.float32),
        scratch_types=[
            pltpu.VMEM((_PER_W,), jnp.int32),
            pltpu.VMEM((_CHUNK, _D), jnp.float32),
            pltpu.SemaphoreType.DMA,
        ],
    )
    def _permute_rows(x_hbm, idx_hbm, out_hbm, idx_v, buf, sem):
        wid = lax.axis_index("s") * _NC + lax.axis_index("c")
        base = wid * _PER_W
        pltpu.sync_copy(idx_hbm.at[pl.ds(base, _PER_W)], idx_v)

        def chunk_body(k, carry):
            off = k * _CHUNK
            pltpu.async_copy(x_hbm.at[idx_v.at[pl.ds(off, _CHUNK)]], buf, sem).wait()
            pltpu.sync_copy(buf, out_hbm.at[pl.ds(base + off, _CHUNK)])
            return carry

        lax.fori_loop(0, _NCHUNK, chunk_body, 0)

    return _permute_rows


def kernel(x):
    xf = x.reshape(_ROWS, _D)
    out = _build_permute()(xf, jnp.asarray(_IDX))
    return out.reshape(_B, _N, _D)


# speedup vs baseline: 2.9623x; 1.1372x over previous
import base64
import functools
import zlib

import jax
import jax.numpy as jnp
import numpy as np
from jax import lax
from jax.experimental import pallas as pl
from jax.experimental.pallas import tpu as pltpu
from jax.experimental.pallas import tpu_sc as plsc

_B = 4
_N = 4096
_D = 2048
_ROWS = _B * _N

_PERM_B85 = (
    "c-jrr0|FR2768E5d8uvNwr$(C-TvCPZQHhO+qSzqhnU_WqhH`p@@V0^sR$*QVL#j#QW7Sp{sDMLlg8brmslg6JB$s<L_Y7(Ri9Qp"
    "!dp-(RWN)ExBLZLDO}ON@jEd|rxm4bS-%{g=E=iJR?2jjxA;VtS0|QhY&6^mr<2vm5!|1y)vM5+P=_Xj2yj~thoPuE+$VGCaXQV9"
    "bxZJOcF_s(6jHcZZiG%3s`)W?6#J_R%51)dUrx@d(lmkUBCf$AwIY1R?ZiE`*xtfQ<x^Qud$T}p@omfux`U12F@(e`)lOW5PFBlo"
    "3*X465SeXT@zzZ;CBrAZmW~Qr`3-VP4G7Ix%Fqyx6J2R>`qH)+XKjC6Q%O`4uaSLea(_%FgYx!>8zobto&1}>1iyF(yBAVG2Kxq="
    "6xGaajKe9{-R;8T^)lK~&2usA0Q5x0Mft@nvcM%F#I40|MOFOUrF8dMT6sa`f@yG0jKEWA9Tf>bhq0oPULy0+@;;wvKsTzE;Wsa?"
    "YM4#phdSz7Lp&HG#_}a7l1^$&7^`-nvveCdh%=Gj_KZhzy_WXB5Y;8OCCDJu9h!jGuV_@-S6@Uo%xGBx4$74J7!Z_`a+i<fHI-xs"
    "yq7fNFa2!&#2#a5P#IoS*EeO{c=1R?Q^jmjkq5=4kwqWAk9>kdP}an;(eXOD-n<}FO%~TuoiGkdw@VC_J^cc;&F?k6X^&7tyd;-d"
    "H1jOX$A?TNIU0{Jk69ewQd;`Sr4Q5}rGVp!(V+_(K&t@I*?1Efz=o6Vs2<OU44fvZF%3P)KA4B+*>kEtdy6}vkEpHN4r$0HU5RJm"
    "^L;&4kWUd)<j*kL{laTqV!njd_1DR8Q&T47`|Jla1~nH~d~UYR^fhNlOgLlT%LaNGxrjUQO01hDYOy@68rT)2uDOSo@>M*Me-bLI"
    "&U%1&gxW%PmIzvisO+B2t_a;mqr1!QzG@)m@$Gax8Utf}XE#{}eHOPgQS@bg6-uBkG7U}|UbDB1qY<LAh_9rIMzc!}F>wo4$|hx%"
    "T?IFs@0TsY16D)bU~5GZH;J~j75re8$|Czi7eWPCIh~yCz)M2@kXYqbR2DOb(G${~r{{mdw-ALVhJo$~AI^4?^(ZcRV~IJ1U$U*T"
    "B+jOD%M-LBzN-q6t#DYkW!J=VQ$`=4?`<pnDtx64{AN(Lb$CsG@)e>z>*pV<#?XzWwU2mA@zlnX!*LquDAVJ?cB#mwy7^3WyZOYn"
    "vJ2sw-_99og_y6xQ~6Y;5{xqu@LFSSX)+d9qql7Yx5QkRMMFK;i6s`(LkiMbE@6d3Trp3!rNz-<n_N~5O(74BA%~dM_Bww7M^Fm6"
    "TJ=L8<XgRgJ=1yY5VQ&p@LS1dUD+QBQ_L^5TkNz~NNL_qHn9uPOE~CaxWaTiFKpV-SpJ<(W=e|+D2}^_p6g<C932@BhAoiE3bIP3"
    "B$UNfd*l?i)Xt;H=`QnA#K8|`DcnkS(}i)N@Lsp^18G9n%Uo24vC^gVP0}x1foE`vUkX{o0eM%K@x%08lgQk0NpUwgCCZDAHXVD<"
    "ULYsehL~oYk4YQaeljQd$+RpV&fplZ%+=LRLQAqoU9??g8XI5Nmq%?TwcBmx{YgJlBs5_gV1kRJKH9ptJf~1aT;SKx6u@qm%4~|@"
    "`=O>E@20Big7b2AT!>A%D9e_jEn&F2&4<x{WKBqb2<y%-nLn_|#bbwkDx4oRV#VD)Uxr>s-&G^|$;OmXaT8KI{3IL1AJLR1W3A~S"
    "u4x5+7rsIbc^K}<{AwL9Mel`<I<tBMMeTL-7fy;!q=|W@uBylKgy^Z>;ZY_pKSz#>p{9w>VA_)V0=et%09hic%2~n7-DIa9i87O|"
    "zM4D1Ht|HH9ItG;t5pR1k|YuBU~(Z&&X@~6nQG~KlA?TxUuRqUg&z6MWSPjIQ<~Yd9_l0e*+;&o{7;O96}qqOM>Fb~XfDfvql%8`"
    "E{;sMh+(`I)bgj~5#1kh@M`L-j8EH;T4)8@!UAvSu7x8CLNb}fC|uBN&_}{Ymro{@J=h;U9Rif3{nZ6rIb=d3Q1ehkRTTY1UOG{1"
    "b|u*_I)oLn9r!(`$Q7Q`E#p^F5s^;yVHv}fFx7Wcb3_$$ja)Ok{3)~#y;D8$cfBBd#Xx@QyKX|5&vWrubXy3lrri))*njL8o{JyJ"
    "qNXgF#(wK9Hb0q&o0<0ZfxRKpqtxNAeGs0jdZwa@NWl#bZP<A|kbSX@#VC4%D%h$fn3epBh>w_@LAs%w{4Vjlhvz1j8(<!i+jK}c"
    "szE6}B0LW{<!<v(T@Bw{53w5FuybgR%P8JiOX7ysY@?ePTDfAfl5a|Gt4VN*J&>i{R5?K>aGm5t6A2chXy}rRFNcw~W{;2J3xij2"
    "fWRHU6h@P~Jhjm*g6#!q<S|uH6%&VD_i#qP<GKBI(woebzu*|1q&{1MhoL$Cl52%u=+-<1yYDKKYxKO2hn}Nrye~h2UnoHrt8TIl"
    "Q2ha4wJXdPlF=QqCEWsT$O>rZU-;}Ov!8B?^FiSw&dO(rO6nK->mQMb;w)-zrjjpou3b!$xss+U-cOr_+~~E)>kimu?43)B_KH5L"
    "HW8sPUvA^-gk*;)1ZB)Dyn}C)j`k9tHP$Zlw^@BP8=Yi=sAvj~V6<o^-=iQ_($%OQP`*vnVll~99Fx~zrNT9L%bizqs0ypuSiRFk"
    "^wj=Ql;)6gSY8$>WT#bR1vyRph8y0p3I3>vW((@AD5l?FR=Gpul^95}&?qpP?L_lMd-~h_Rt-p1xG6{B?Ru*Cs4v-Xc8_N2KCBnb"
    "ttK06W!=PIv!B%?yIqzL`Aip^P<=JM^)?tK`jFuBiY{!1mgFINjLw9PK8bD{KATP9w24m&t9Ics-fW)o4`e8=;`ZzPA)-v}ckwF}"
    ";c>2ztRH5(j3|c3VulGQzBx_TE5gB)rCaG5P2Cg}S*ODr{SG$~&Eloty2=~o!4cn540n#@M1y4mjL{LfBUED@!$VQk&Z0zL7BkQ="
    "yWY)4P85&{WMVdj8pmmcup5t7l_9Eq2$%2y-q$sj7fobe!W2*mjfEb3oUW)U)5u2S9<rY3qg$f&qzP~2>e2N+p6?*e@QnPnuHxI-"
    "Rf4nmXge!PxAC+(A`a?N*a6q{Yt+f7#C54)DRg2SjdkQwFLir(8GA@R!B2dB{E}voX$+R}^eyXccGl75gITN>Pp$LwbG(geE}yy>"
    "thB1nDQc{TspNv|NntMP?EB(ksE2v2>#@G<2#bqn$R4m)MT9bJowy}(+V3Ghz6d2zITsr)(0}5H8EO9M1!N&9%d@elJg@Ac*TWCn"
    "3pG%!d=0#hCPz<rLMi2B7uh!T`DHiW!5?B3a7~?!43y_ZJMvO)P*s#LhiEDJ%oIjv<aauoO_C$jbh%SKCb7gfNN3Bsb2Ku!N{*34"
    "a)xT)ve}ZlpGq&L+FEV_&qBHxi{FuvwyF6ll5mV0isCl9$ELm=<-YqHF1JAlsZ*Z&$v9t#LR({PKFZC$jjRciP;P$Ex8qfiK>zGz"
    "QwOf8uHu7U=Z5*lA-X7sk+>q?!V5Z0*T#?NCy|QR@vm)4_@G*uEh;YWtZJZXWGZ^;2D2A@k9`+@+1z-wedc2Es%)0UEDdjCv)h$="
    "m;8dJi?-+<w1f><=wYmCc&6UtOZ=%!?b7*-@SMK)dqi4P+E+D$(O;Zfq^1j1EIrwDkg-BVmc_?*rNMDb59x1s2kC>O+XCU7j$*3o"
    "TILq*3&lfITUsVCh5c?inf%9h>I*s&{pw?rSZWKcF7J?LDz?5$&%he~$DFriP&QH0XS73EV!KMMm)0MlUF~w3k#vIALPC3(Y2Kn3"
    "c&KmA#<1uzj@hANqYr#;_^SG&^dy`7CC2d@?u)<Y+v5E`BD?R?=#`KeHKDu182!fYve)V5@I>FEv!M#T&qwJ?y0F{BhUz5jpqh-D"
    "$_s9h9wsC5*Zhra$=BM~q%*Fl7qQ9g8&B*f+RijBStF|Gf_|(0M;}2|Cw&)uK<~r7RCQm`{;^A-peVw->I||Ox?xlLoS~IkVA|^;"
    "qz!6}zOtXLCQkq>S$jM)tac+sCtL@=f`;LVN*Lmi@wAiPB(w2yVvKo6|DemHf<75$@QVJGIfs|CSoSH(g@ww+c8V^c8RQbla8y;0"
    ")pkYTs4B#Z`h@xid8Est^nN<;q&k@h`b%hvj>_pOo++Vc^KX*kZmt$S4&MC-gruXf+&)=GuEP~zPAIDrxrKbSJTD)b!)6luBA(;B"
    "vLbrMPKQHono6!J;nOH<NJu+?r*%*sU7lGz+9pADcsbINwZJ1)Lw?z|)kWl4kwMJVNrJ(rU1FDm7B?%+=J3hPq_t!VR2ipKb=)yg"
    "JrtMPEz{Ce6K~}u(TN)0h3C_I?LTwRT({%dJT#Yo=85bg8;kwN-tx(Og>2!XppEPedhZl1E;58U=(XDo_swPXneC;g$OhERJrk8t"
    "0vt(9a-Go;5ydsfyV3dZK?r_GMH5YR2Yph_Cn<bQy3!9v5$M0r1kc2y(Qg0TKWBgGaaN4iwQ)!-J{O9zxUR041+8rbEAS1~CtR^j"
    "d0AC1+=2{rp-qSKu%+ZY%A)hRpZ+7MZL8q4Gy;i(Ci$agEhLcz{ClPSr?AkK#}~tUzL5^J(_~U@^lbCl@1S+qZZ*i4A?0Wn7l~~0"
    "=ODFcz*2=n;v1dj(wmF2u+4{)sQ3`yR;Tr33iLNPKY?Cj9zOXsY=Dz|hP#1#vL9>^sVWBI3?#QLDr(T1-l{}myANg<ZX#-;TbkiY"
    "uDGtCUh*JE+2=MDr1j<9MKQ-F&<YI;Z$sCR-K{n~a0InkeC40iNz{e5CLcmcn}Fujk$DNZk97!v^l(dYc5(-XnN<2^NX;9<ZrhMf"
    "cY__%(cEn)OdiuHwhtWlHAz{UIdlul%zj?O{-%9Jj1W1@RI7Csi%5UIQ(j<Kag7kkjbl?7<2lU-zn`@*t<@ltBXl7j99N^@imFJj"
    "U}R209(mK|B?DwueP3p#Uv*+i)kuGlWj6<PIh7EvqvPxgcbNV5MP1ae*5_k`aT$ET&ksvk19Vq>;&<%}`=9+`_JG#M{7RJ0B;X4}"
    "Y_S(#!|muc)X+~<d(8VV-zA6n@(;epvD)Ao+Z8M?3{^>NZaoLa@TI1g-9xYH#XL31rBCBxc(R|4cCvZ;9o#|_NES93l#e5;^1fuA"
    "KftPy+@zOZjI%*pxd|mFa~%`;&3I_2m(tPvKYT#_04hrBLM)fPsWY=ha-h12Ys>7Okug3Kdxna**RVSLryk<VXudjV^6PZupiSUE"
    "Kw5TKPA5lfS9b#%n(bnUo`E0Ro}w$;6>5;y2HP{V66r5K;U%^P^waf1afl)&vXlBCX^hv(Pb9X#9%AYU{yyH#6Oco=6(4IE{F5C}"
    "+TcS<d(q!^8s#w!!W7?Doinf1JT}*4l<`A9R@-b=8)zZb+7)wk$=+bma0}3ZWW{BmzaA+asY2eW6RMrAU=pE%Xgm7NzM)Y(y(%x`"
    "&@|y5DQ#xyVx%|QL;lL_qK7J>ir59Xq3Wf7`5bJX$SU8Pf9|b~XRDi&c&kYxVyeihv)pKx!x)kfB{2&?xL542?#WNf5_X6@8v+gC"
    "c&HgF^RZ;SeC{@xb)llX8K%)(v<;p}%A0b~8;=P`*-EkkUuE-kM3)n4^Gmd+iG&jAH@q`Dqq4wyyNfjy=WK0%kaZ(U<hO^#E%t&n"
    "Bk#p%e@_09PgO;iheY!u>0id&A2QEmcID(ZH3#R!x6~V&C^)FEtD}wNB7aNo;5xP%S|rBE5wIV9CmPDDxqg6-%O03EdN5rS`iPJ4"
    "UFIMQM5gc=zPK`?0E(eLuyG27@@Oah&OiFHkcyqBt7vDESs%dbAOcS%o|{EvFX~PQ*v#^<8m+h4jiQek82-49GzA~(4$~9R-2dkX"
    "iYPJ(ob}D@AX?eB)VVxkCw(OL0EUTrwjHbrxm0gIhnM0<@pZjhPuFMRvUl!}zJ%KQ33e|(BMR#s;tq=iEy#60mi87Y@MrQ(9Hf8A"
    "A)lDc4$;FR{=t?}>*)@^!={$sZ81nj54eKlrMMA}lbR;2tDy(UOFmD?!$;GqG?{<EABe0h4j<;cIANcQS7;y_tc#Q0va2euuevuz"
    "(n=_c8i!B&QK%wcX!58@Jhn}vKG4W+i>@nYnIfnoG(pQ{J-@`wRuknSh#5Ykk+_a*L?(zurZ|g*$B~K9N@Ne^Of~#Qx5Lr&c3hW_"
    "HI>A7F~@YHEo8Fr{|;$v9-sB358MdXm__F!{7ipbp5QU~C)&ps(`!{Nty}_oQ|)yN*$TTECnbYnw$0&Q;reQ$uFaN))B2JsVsdzn"
    "W3t+68jnhrq3Ak~D+(L<YLbiWS9@G?d{eZfBi(aS-~X0z{Wx@1kCRQq5PT>6VvC`jX~d_x8a%DZN>9N94I(-J$5Xioq5$s<XJu|k"
    "u9K>LhM9j*R2_xYyt!T~U)eu22I?zYqk?WK`=?IYU%H1~rtavLI5L{Ss@RWwB-~~hSv|a2eh#PTS~7wa5;??2brfY4@8|_tmo3%_"
    "MP2mRt@B;UZg!r$L%mfbe+Ol7FHH^=pEpM*X|+&W6JHk=(q(c#PC|=?{P0-3_D6kma}XU1pT!tABMjG5b$*tP7jW<7?U2=$lx1-f"
    "H48n!XI%>_V3mL1--zl^$^5`CL~7Q+<na;YGB|`T(#JS1WKbJnh#h28q8@ZHb2gR<IF9HPnqnJ*jl}D_sAhe@yhPXu#eEy`9nXMu"
    "=r`%@B9aO4l$GQS(HVA~6~|WG*Ns&vHe61pLwGB;92J$%$Szt)KXjFBdOHBG_H)S}-xE@@xjLh&4L?zNQip8ATSZdW4pr4pXf@a-"
    "H`8l!7kQ1_glVX($jKIq=4!5(Pdnl(F1@>DT3I7w&}_J)=;{|jUUkN;GE4OrJ<tzQjp<L@T2IAS=?1^cU4~?~p029elLv^{MwZKn"
    "I)^!q-?#;K5Ff}(sS>`CTEh2}+v01;j{89~T%OLB+d~WW!#5;-RYj83RmYvfKl~qxr&dtKR?`ZsrhAEg(srtSm~1EOg0`)ErT4p@"
    "_AYD8hXCRc(Q&m2zmiGtJsq2EaVPX=H-~ugQQY9SNPN5p&4)GaiCd2csP2A^N@}m;L85e6p_2Oe;tTs7x|n~yAD@Kl3m#5`G8s@K"
    "dMmsLDd}GJmoE?5Tw<LHr4tFlJ64@tHB;<hlLLWYs?MNTvZY%Y2EcQfN;*YkB>6;C<4-k)KD4k{iJw4T9~pPY+g&f3Q)FPzWEz&+"
    "jnqd75x>HB*aa0xe{)M_hu^}`6X-0PW=r5w#F(KX6*=w_v3Mc4#;&GFZoZOLFw1vvMiiu(a8?vY=ck~%lkvC|8bJm@T3GD1;AFfR"
    "okytoE55O_rZX?88nP>90C{ZE%R<zs0%9WnVAk_HHj`^i^5A9RE6PaY;pP6GSmCy*d^{<S#^dpAYz-NNi<0TSKH6`hgvS1+tLs`o"
    "SF=zgceUs&`N+i%bCt*Cd<u)n7nNO~B@xwZ|5>I8B&3uP!akA0j8YyI@sS~kpCs;v!{nuY=6kxsVG7USe}oBmiOv<C;>V#L=?cBr"
    "Q##FLB}H*Hxlm^`fbQ^eEH6$*vV^hnHX0#MgrBkl+aas^eQGe@rkj&><S**$^CE3!NI@HklWw%BqZ`OHbhDkn&*RrJp&g`~`DWrM"
    "87;P$Chn0+4Gmp-)<V~^pHNj&-Hi{^*_4n~rNuvCFj+*W;o3AQEsdVpHhjB1=I-0r<{~*2F2E}L*IlNu(0{a1;5MFH<4=+$<UFfS"
    "W2@D;y%MrjSScDC;eV=`<dtvgKGL43x$DidLIH9GJu!XIc-7n7W1C>Eyovg`IWirzg57c?sYOQ6p1L-Q7=}1!s-jL}zs(Fw{TE(L"
    "4TZAuKqz3xqM4$*?k0-)N#sBBO&{a^Ft9GLRsSbb(VcD-DWb27&L%S37z&v$IFo<nKn|yOeJ+zp?{if`Q+1Pe@s(&@^&Ksgot&|S"
    "WiBb%&X71{GLh^FnTT(ZcT{(~Uvw8Y&|W`;P7M!r0m^JU7DG>_>)0>znbwC#><c)ZUO!~f2%|%3eqWfaK|O<#b5v^ehO9BxY;|Qq"
    "Lpct<kq9UuOXsf2JkXI{hx%kAzM*fjWI8(Si%!|8JT1SdqoYP9q3`1cx&Csth-(JwXwZRH5Gln3*$5)qj^qf;3#Zv)+5ir~1T+jz"
    "=^?%VPRYNSnXCbSYtHii?jPz!hKrbXCQra-_}3vbkD^kjE360VN9vdrKCg{yKbrUSv5&3}i0)=+*rz+&wmheQhBIR;+oLaLx~;Ar"
    ">-Mf}sBVY*m39p7p`+^de2MbrliY((xO3qZdCZU5w`6hHKr-S|BAb!oxXCOExgi3<<nV+YM}`-~lw=YG%sQP%9OY5vNWNT6Q7vt3"
    "d57+m1z<B;#hTMiw3o}kSGtG19E!unvc%@0-DZRx$v(13CXsp%7t9N@$;D9%=y6@0FXxx_JKvSR^(A;I{{vOi<IQ279#%>nPTK`6"
    "H?5_Tp@rx;pXr8@ZLXHMEN7D`WQ4Umfk1kbTBjngRr)b3Hdpx<S<3B)G=43Mja$+4un`qSTSJ{t!gWwL;R5M~FNhX=4K!uH<P@EW"
    "7xMq<F=~^|sp5OEgS|m(@NYCnuNI5&c)LR%)6MjFGT&Ba)50aPLEaJBc@Eu0E%wD&-LNnGG@a~Co)o^jr(qLYh<~~9KC;}%^VzKC"
    "37V(QoAvB2`p;Ei^UVP>i2V?E%u7`(yfy>P1NDr1|1gxG2hbvv1A3!(W<HxCZpb<!8;uT8{Ym+NOvf*94Aa(6)V*zU7zigwbavit"
    "SEt1r6@y$rS4B-0P5xo&^f?ya9SvR45Ve-?(mT+7bj}?V%XNC%8)n)JaMKioA3R&w$D7Efq7f?1w(#X{3_WIwkSN5ur{+AbEn<;b"
    "YOfw@U&xb2^W!$6nF<qFAA8Qf39U#>*Iqs{r_3_XLK0Ddu0>y6HnfH!_7YF@%WxJv%0KiO@DCb~ws8k}em8&@!jr6L^;r?N(R{UW"
    "!thW)KlPo(J=x#)!o9*^aoZ=te@ztplq?tD*-tzZZj)oEZ}>$j%a~{dKL`(KX*`riVbACy|3m&~5~JIummCc_><K@SP8KuldlD6%"
    "^H=<T%q?(v"
)
_PERM = np.frombuffer(
    zlib.decompress(base64.b85decode(_PERM_B85)), dtype="<u2"
).astype(np.int64)
_IDX = (_PERM[None, :] + _N * np.arange(_B)[:, None]).reshape(-1).astype(np.int32)

_NC = 2
_NS = 16
_NW = _NC * _NS
_PER_W = _ROWS // _NW
_CHUNK = 16
_NCHUNK = _PER_W // _CHUNK
_NBUF = 2

@functools.cache
def _build_permute():
    mesh = plsc.VectorSubcoreMesh(core_axis_name="c", subcore_axis_name="s")

    @functools.partial(
        pl.kernel,
        mesh=mesh,
        out_type=jax.ShapeDtypeStruct((_ROWS, _D), jnp.float32),
        scratch_types=[
            pltpu.VMEM((_PER_W,), jnp.int32),
            pltpu.VMEM((_CHUNK, _D), jnp.float32),
            pltpu.VMEM((_CHUNK, _D), jnp.float32),
            pltpu.SemaphoreType.DMA,
            pltpu.SemaphoreType.DMA,
            pltpu.SemaphoreType.DMA,
            pltpu.SemaphoreType.DMA,
        ],
    )
    def _permute_rows(x_hbm, idx_hbm, out_hbm, idx_v, buf0, buf1, gs0, gs1, ws0, ws1):
        wid = lax.axis_index("s") * _NC + lax.axis_index("c")
        base = wid * _PER_W
        pltpu.sync_copy(idx_hbm.at[pl.ds(base, _PER_W)], idx_v)

        bufs, gsems, wsems = (buf0, buf1), (gs0, gs1), (ws0, ws1)

        def gather_desc(k, b):
            src = x_hbm.at[idx_v.at[pl.ds(k * _CHUNK, _CHUNK)]]
            return pltpu.make_async_copy(src, bufs[b], gsems[b])

        def write_desc(k, b):
            dst = out_hbm.at[pl.ds(base + k * _CHUNK, _CHUNK)]
            return pltpu.make_async_copy(bufs[b], dst, wsems[b])

        for b in range(_NBUF):
            gather_desc(b, b).start()

        def body(g, carry):
            k0 = g * _NBUF
            for b in range(_NBUF):
                gather_desc(k0 + b, b).wait()
                write_desc(k0 + b, b).start()
            for b in range(_NBUF):
                k = k0 + b
                write_desc(k, b).wait()

                @pl.when(k + _NBUF < _NCHUNK)
                def _(b=b, k=k):
                    gather_desc(k + _NBUF, b).start()

            return carry

        lax.fori_loop(0, _NCHUNK // _NBUF, body, 0)

    return _permute_rows


def kernel(x):
    xf = x.reshape(_ROWS, _D)
    out = _build_permute()(xf, jnp.asarray(_IDX))
    return out.reshape(_B, _N, _D)


# speedup vs baseline: 2.9683x; 1.0020x over previous
import base64
import functools
import zlib

import jax
import jax.numpy as jnp
import numpy as np
from jax import lax
from jax.experimental import pallas as pl
from jax.experimental.pallas import tpu as pltpu
from jax.experimental.pallas import tpu_sc as plsc

_B = 4
_N = 4096
_D = 2048
_ROWS = _B * _N

_PERM_B85 = (
    "c-jrr0|FR2768E5d8uvNwr$(C-TvCPZQHhO+qSzqhnU_WqhH`p@@V0^sR$*QVL#j#QW7Sp{sDMLlg8brmslg6JB$s<L_Y7(Ri9Qp"
    "!dp-(RWN)ExBLZLDO}ON@jEd|rxm4bS-%{g=E=iJR?2jjxA;VtS0|QhY&6^mr<2vm5!|1y)vM5+P=_Xj2yj~thoPuE+$VGCaXQV9"
    "bxZJOcF_s(6jHcZZiG%3s`)W?6#J_R%51)dUrx@d(lmkUBCf$AwIY1R?ZiE`*xtfQ<x^Qud$T}p@omfux`U12F@(e`)lOW5PFBlo"
    "3*X465SeXT@zzZ;CBrAZmW~Qr`3-VP4G7Ix%Fqyx6J2R>`qH)+XKjC6Q%O`4uaSLea(_%FgYx!>8zobto&1}>1iyF(yBAVG2Kxq="
    "6xGaajKe9{-R;8T^)lK~&2usA0Q5x0Mft@nvcM%F#I40|MOFOUrF8dMT6sa`f@yG0jKEWA9Tf>bhq0oPULy0+@;;wvKsTzE;Wsa?"
    "YM4#phdSz7Lp&HG#_}a7l1^$&7^`-nvveCdh%=Gj_KZhzy_WXB5Y;8OCCDJu9h!jGuV_@-S6@Uo%xGBx4$74J7!Z_`a+i<fHI-xs"
    "yq7fNFa2!&#2#a5P#IoS*EeO{c=1R?Q^jmjkq5=4kwqWAk9>kdP}an;(eXOD-n<}FO%~TuoiGkdw@VC_J^cc;&F?k6X^&7tyd;-d"
    "H1jOX$A?TNIU0{Jk69ewQd;`Sr4Q5}rGVp!(V+_(K&t@I*?1Efz=o6Vs2<OU44fvZF%3P)KA4B+*>kEtdy6}vkEpHN4r$0HU5RJm"
    "^L;&4kWUd)<j*kL{laTqV!njd_1DR8Q&T47`|Jla1~nH~d~UYR^fhNlOgLlT%LaNGxrjUQO01hDYOy@68rT)2uDOSo@>M*Me-bLI"
    "&U%1&gxW%PmIzvisO+B2t_a;mqr1!QzG@)m@$Gax8Utf}XE#{}eHOPgQS@bg6-uBkG7U}|UbDB1qY<LAh_9rIMzc!}F>wo4$|hx%"
    "T?IFs@0TsY16D)bU~5GZH;J~j75re8$|Czi7eWPCIh~yCz)M2@kXYqbR2DOb(G${~r{{mdw-ALVhJo$~AI^4?^(ZcRV~IJ1U$U*T"
    "B+jOD%M-LBzN-q6t#DYkW!J=VQ$`=4?`<pnDtx64{AN(Lb$CsG@)e>z>*pV<#?XzWwU2mA@zlnX!*LquDAVJ?cB#mwy7^3WyZOYn"
    "vJ2sw-_99og_y6xQ~6Y;5{xqu@LFSSX)+d9qql7Yx5QkRMMFK;i6s`(LkiMbE@6d3Trp3!rNz-<n_N~5O(74BA%~dM_Bww7M^Fm6"
    "TJ=L8<XgRgJ=1yY5VQ&p@LS1dUD+QBQ_L^5TkNz~NNL_qHn9uPOE~CaxWaTiFKpV-SpJ<(W=e|+D2}^_p6g<C932@BhAoiE3bIP3"
    "B$UNfd*l?i)Xt;H=`QnA#K8|`DcnkS(}i)N@Lsp^18G9n%Uo24vC^gVP0}x1foE`vUkX{o0eM%K@x%08lgQk0NpUwgCCZDAHXVD<"
    "ULYsehL~oYk4YQaeljQd$+RpV&fplZ%+=LRLQAqoU9??g8XI5Nmq%?TwcBmx{YgJlBs5_gV1kRJKH9ptJf~1aT;SKx6u@qm%4~|@"
    "`=O>E@20Big7b2AT!>A%D9e_jEn&F2&4<x{WKBqb2<y%-nLn_|#bbwkDx4oRV#VD)Uxr>s-&G^|$;OmXaT8KI{3IL1AJLR1W3A~S"
    "u4x5+7rsIbc^K}<{AwL9Mel`<I<tBMMeTL-7fy;!q=|W@uBylKgy^Z>;ZY_pKSz#>p{9w>VA_)V0=et%09hic%2~n7-DIa9i87O|"
    "zM4D1Ht|HH9ItG;t5pR1k|YuBU~(Z&&X@~6nQG~KlA?TxUuRqUg&z6MWSPjIQ<~Yd9_l0e*+;&o{7;O96}qqOM>Fb~XfDfvql%8`"
    "E{;sMh+(`I)bgj~5#1kh@M`L-j8EH;T4)8@!UAvSu7x8CLNb}fC|uBN&_}{Ymro{@J=h;U9Rif3{nZ6rIb=d3Q1ehkRTTY1UOG{1"
    "b|u*_I)oLn9r!(`$Q7Q`E#p^F5s^;yVHv}fFx7Wcb3_$$ja)Ok{3)~#y;D8$cfBBd#Xx@QyKX|5&vWrubXy3lrri))*njL8o{JyJ"
    "qNXgF#(wK9Hb0q&o0<0ZfxRKpqtxNAeGs0jdZwa@NWl#bZP<A|kbSX@#VC4%D%h$fn3epBh>w_@LAs%w{4Vjlhvz1j8(<!i+jK}c"
    "szE6}B0LW{<!<v(T@Bw{53w5FuybgR%P8JiOX7ysY@?ePTDfAfl5a|Gt4VN*J&>i{R5?K>aGm5t6A2chXy}rRFNcw~W{;2J3xij2"
    "fWRHU6h@P~Jhjm*g6#!q<S|uH6%&VD_i#qP<GKBI(woebzu*|1q&{1MhoL$Cl52%u=+-<1yYDKKYxKO2hn}Nrye~h2UnoHrt8TIl"
    "Q2ha4wJXdPlF=QqCEWsT$O>rZU-;}Ov!8B?^FiSw&dO(rO6nK->mQMb;w)-zrjjpou3b!$xss+U-cOr_+~~E)>kimu?43)B_KH5L"
    "HW8sPUvA^-gk*;)1ZB)Dyn}C)j`k9tHP$Zlw^@BP8=Yi=sAvj~V6<o^-=iQ_($%OQP`*vnVll~99Fx~zrNT9L%bizqs0ypuSiRFk"
    "^wj=Ql;)6gSY8$>WT#bR1vyRph8y0p3I3>vW((@AD5l?FR=Gpul^95}&?qpP?L_lMd-~h_Rt-p1xG6{B?Ru*Cs4v-Xc8_N2KCBnb"
    "ttK06W!=PIv!B%?yIqzL`Aip^P<=JM^)?tK`jFuBiY{!1mgFINjLw9PK8bD{KATP9w24m&t9Ics-fW)o4`e8=;`ZzPA)-v}ckwF}"
    ";c>2ztRH5(j3|c3VulGQzBx_TE5gB)rCaG5P2Cg}S*ODr{SG$~&Eloty2=~o!4cn540n#@M1y4mjL{LfBUED@!$VQk&Z0zL7BkQ="
    "yWY)4P85&{WMVdj8pmmcup5t7l_9Eq2$%2y-q$sj7fobe!W2*mjfEb3oUW)U)5u2S9<rY3qg$f&qzP~2>e2N+p6?*e@QnPnuHxI-"
    "Rf4nmXge!PxAC+(A`a?N*a6q{Yt+f7#C54)DRg2SjdkQwFLir(8GA@R!B2dB{E}voX$+R}^eyXccGl75gITN>Pp$LwbG(geE}yy>"
    "thB1nDQc{TspNv|NntMP?EB(ksE2v2>#@G<2#bqn$R4m)MT9bJowy}(+V3Ghz6d2zITsr)(0}5H8EO9M1!N&9%d@elJg@Ac*TWCn"
    "3pG%!d=0#hCPz<rLMi2B7uh!T`DHiW!5?B3a7~?!43y_ZJMvO)P*s#LhiEDJ%oIjv<aauoO_C$jbh%SKCb7gfNN3Bsb2Ku!N{*34"
    "a)xT)ve}ZlpGq&L+FEV_&qBHxi{FuvwyF6ll5mV0isCl9$ELm=<-YqHF1JAlsZ*Z&$v9t#LR({PKFZC$jjRciP;P$Ex8qfiK>zGz"
    "QwOf8uHu7U=Z5*lA-X7sk+>q?!V5Z0*T#?NCy|QR@vm)4_@G*uEh;YWtZJZXWGZ^;2D2A@k9`+@+1z-wedc2Es%)0UEDdjCv)h$="
    "m;8dJi?-+<w1f><=wYmCc&6UtOZ=%!?b7*-@SMK)dqi4P+E+D$(O;Zfq^1j1EIrwDkg-BVmc_?*rNMDb59x1s2kC>O+XCU7j$*3o"
    "TILq*3&lfITUsVCh5c?inf%9h>I*s&{pw?rSZWKcF7J?LDz?5$&%he~$DFriP&QH0XS73EV!KMMm)0MlUF~w3k#vIALPC3(Y2Kn3"
    "c&KmA#<1uzj@hANqYr#;_^SG&^dy`7CC2d@?u)<Y+v5E`BD?R?=#`KeHKDu182!fYve)V5@I>FEv!M#T&qwJ?y0F{BhUz5jpqh-D"
    "$_s9h9wsC5*Zhra$=BM~q%*Fl7qQ9g8&B*f+RijBStF|Gf_|(0M;}2|Cw&)uK<~r7RCQm`{;^A-peVw->I||Ox?xlLoS~IkVA|^;"
    "qz!6}zOtXLCQkq>S$jM)tac+sCtL@=f`;LVN*Lmi@wAiPB(w2yVvKo6|DemHf<75$@QVJGIfs|CSoSH(g@ww+c8V^c8RQbla8y;0"
    ")pkYTs4B#Z`h@xid8Est^nN<;q&k@h`b%hvj>_pOo++Vc^KX*kZmt$S4&MC-gruXf+&)=GuEP~zPAIDrxrKbSJTD)b!)6luBA(;B"
    "vLbrMPKQHono6!J;nOH<NJu+?r*%*sU7lGz+9pADcsbINwZJ1)Lw?z|)kWl4kwMJVNrJ(rU1FDm7B?%+=J3hPq_t!VR2ipKb=)yg"
    "JrtMPEz{Ce6K~}u(TN)0h3C_I?LTwRT({%dJT#Yo=85bg8;kwN-tx(Og>2!XppEPedhZl1E;58U=(XDo_swPXneC;g$OhERJrk8t"
    "0vt(9a-Go;5ydsfyV3dZK?r_GMH5YR2Yph_Cn<bQy3!9v5$M0r1kc2y(Qg0TKWBgGaaN4iwQ)!-J{O9zxUR041+8rbEAS1~CtR^j"
    "d0AC1+=2{rp-qSKu%+ZY%A)hRpZ+7MZL8q4Gy;i(Ci$agEhLcz{ClPSr?AkK#}~tUzL5^J(_~U@^lbCl@1S+qZZ*i4A?0Wn7l~~0"
    "=ODFcz*2=n;v1dj(wmF2u+4{)sQ3`yR;Tr33iLNPKY?Cj9zOXsY=Dz|hP#1#vL9>^sVWBI3?#QLDr(T1-l{}myANg<ZX#-;TbkiY"
    "uDGtCUh*JE+2=MDr1j<9MKQ-F&<YI;Z$sCR-K{n~a0InkeC40iNz{e5CLcmcn}Fujk$DNZk97!v^l(dYc5(-XnN<2^NX;9<ZrhMf"
    "cY__%(cEn)OdiuHwhtWlHAz{UIdlul%zj?O{-%9Jj1W1@RI7Csi%5UIQ(j<Kag7kkjbl?7<2lU-zn`@*t<@ltBXl7j99N^@imFJj"
    "U}R209(mK|B?DwueP3p#Uv*+i)kuGlWj6<PIh7EvqvPxgcbNV5MP1ae*5_k`aT$ET&ksvk19Vq>;&<%}`=9+`_JG#M{7RJ0B;X4}"
    "Y_S(#!|muc)X+~<d(8VV-zA6n@(;epvD)Ao+Z8M?3{^>NZaoLa@TI1g-9xYH#XL31rBCBxc(R|4cCvZ;9o#|_NES93l#e5;^1fuA"
    "KftPy+@zOZjI%*pxd|mFa~%`;&3I_2m(tPvKYT#_04hrBLM)fPsWY=ha-h12Ys>7Okug3Kdxna**RVSLryk<VXudjV^6PZupiSUE"
    "Kw5TKPA5lfS9b#%n(bnUo`E0Ro}w$;6>5;y2HP{V66r5K;U%^P^waf1afl)&vXlBCX^hv(Pb9X#9%AYU{yyH#6Oco=6(4IE{F5C}"
    "+TcS<d(q!^8s#w!!W7?Doinf1JT}*4l<`A9R@-b=8)zZb+7)wk$=+bma0}3ZWW{BmzaA+asY2eW6RMrAU=pE%Xgm7NzM)Y(y(%x`"
    "&@|y5DQ#xyVx%|QL;lL_qK7J>ir59Xq3Wf7`5bJX$SU8Pf9|b~XRDi&c&kYxVyeihv)pKx!x)kfB{2&?xL542?#WNf5_X6@8v+gC"
    "c&HgF^RZ;SeC{@xb)llX8K%)(v<;p}%A0b~8;=P`*-EkkUuE-kM3)n4^Gmd+iG&jAH@q`Dqq4wyyNfjy=WK0%kaZ(U<hO^#E%t&n"
    "Bk#p%e@_09PgO;iheY!u>0id&A2QEmcID(ZH3#R!x6~V&C^)FEtD}wNB7aNo;5xP%S|rBE5wIV9CmPDDxqg6-%O03EdN5rS`iPJ4"
    "UFIMQM5gc=zPK`?0E(eLuyG27@@Oah&OiFHkcyqBt7vDESs%dbAOcS%o|{EvFX~PQ*v#^<8m+h4jiQek82-49GzA~(4$~9R-2dkX"
    "iYPJ(ob}D@AX?eB)VVxkCw(OL0EUTrwjHbrxm0gIhnM0<@pZjhPuFMRvUl!}zJ%KQ33e|(BMR#s;tq=iEy#60mi87Y@MrQ(9Hf8A"
    "A)lDc4$;FR{=t?}>*)@^!={$sZ81nj54eKlrMMA}lbR;2tDy(UOFmD?!$;GqG?{<EABe0h4j<;cIANcQS7;y_tc#Q0va2euuevuz"
    "(n=_c8i!B&QK%wcX!58@Jhn}vKG4W+i>@nYnIfnoG(pQ{J-@`wRuknSh#5Ykk+_a*L?(zurZ|g*$B~K9N@Ne^Of~#Qx5Lr&c3hW_"
    "HI>A7F~@YHEo8Fr{|;$v9-sB358MdXm__F!{7ipbp5QU~C)&ps(`!{Nty}_oQ|)yN*$TTECnbYnw$0&Q;reQ$uFaN))B2JsVsdzn"
    "W3t+68jnhrq3Ak~D+(L<YLbiWS9@G?d{eZfBi(aS-~X0z{Wx@1kCRQq5PT>6VvC`jX~d_x8a%DZN>9N94I(-J$5Xioq5$s<XJu|k"
    "u9K>LhM9j*R2_xYyt!T~U)eu22I?zYqk?WK`=?IYU%H1~rtavLI5L{Ss@RWwB-~~hSv|a2eh#PTS~7wa5;??2brfY4@8|_tmo3%_"
    "MP2mRt@B;UZg!r$L%mfbe+Ol7FHH^=pEpM*X|+&W6JHk=(q(c#PC|=?{P0-3_D6kma}XU1pT!tABMjG5b$*tP7jW<7?U2=$lx1-f"
    "H48n!XI%>_V3mL1--zl^$^5`CL~7Q+<na;YGB|`T(#JS1WKbJnh#h28q8@ZHb2gR<IF9HPnqnJ*jl}D_sAhe@yhPXu#eEy`9nXMu"
    "=r`%@B9aO4l$GQS(HVA~6~|WG*Ns&vHe61pLwGB;92J$%$Szt)KXjFBdOHBG_H)S}-xE@@xjLh&4L?zNQip8ATSZdW4pr4pXf@a-"
    "H`8l!7kQ1_glVX($jKIq=4!5(Pdnl(F1@>DT3I7w&}_J)=;{|jUUkN;GE4OrJ<tzQjp<L@T2IAS=?1^cU4~?~p029elLv^{MwZKn"
    "I)^!q-?#;K5Ff}(sS>`CTEh2}+v01;j{89~T%OLB+d~WW!#5;-RYj83RmYvfKl~qxr&dtKR?`ZsrhAEg(srtSm~1EOg0`)ErT4p@"
    "_AYD8hXCRc(Q&m2zmiGtJsq2EaVPX=H-~ugQQY9SNPN5p&4)GaiCd2csP2A^N@}m;L85e6p_2Oe;tTs7x|n~yAD@Kl3m#5`G8s@K"
    "dMmsLDd}GJmoE?5Tw<LHr4tFlJ64@tHB;<hlLLWYs?MNTvZY%Y2EcQfN;*YkB>6;C<4-k)KD4k{iJw4T9~pPY+g&f3Q)FPzWEz&+"
    "jnqd75x>HB*aa0xe{)M_hu^}`6X-0PW=r5w#F(KX6*=w_v3Mc4#;&GFZoZOLFw1vvMiiu(a8?vY=ck~%lkvC|8bJm@T3GD1;AFfR"
    "okytoE55O_rZX?88nP>90C{ZE%R<zs0%9WnVAk_HHj`^i^5A9RE6PaY;pP6GSmCy*d^{<S#^dpAYz-NNi<0TSKH6`hgvS1+tLs`o"
    "SF=zgceUs&`N+i%bCt*Cd<u)n7nNO~B@xwZ|5>I8B&3uP!akA0j8YyI@sS~kpCs;v!{nuY=6kxsVG7USe}oBmiOv<C;>V#L=?cBr"
    "Q##FLB}H*Hxlm^`fbQ^eEH6$*vV^hnHX0#MgrBkl+aas^eQGe@rkj&><S**$^CE3!NI@HklWw%BqZ`OHbhDkn&*RrJp&g`~`DWrM"
    "87;P$Chn0+4Gmp-)<V~^pHNj&-Hi{^*_4n~rNuvCFj+*W;o3AQEsdVpHhjB1=I-0r<{~*2F2E}L*IlNu(0{a1;5MFH<4=+$<UFfS"
    "W2@D;y%MrjSScDC;eV=`<dtvgKGL43x$DidLIH9GJu!XIc-7n7W1C>Eyovg`IWirzg57c?sYOQ6p1L-Q7=}1!s-jL}zs(Fw{TE(L"
    "4TZAuKqz3xqM4$*?k0-)N#sBBO&{a^Ft9GLRsSbb(VcD-DWb27&L%S37z&v$IFo<nKn|yOeJ+zp?{if`Q+1Pe@s(&@^&Ksgot&|S"
    "WiBb%&X71{GLh^FnTT(ZcT{(~Uvw8Y&|W`;P7M!r0m^JU7DG>_>)0>znbwC#><c)ZUO!~f2%|%3eqWfaK|O<#b5v^ehO9BxY;|Qq"
    "Lpct<kq9UuOXsf2JkXI{hx%kAzM*fjWI8(Si%!|8JT1SdqoYP9q3`1cx&Csth-(JwXwZRH5Gln3*$5)qj^qf;3#Zv)+5ir~1T+jz"
    "=^?%VPRYNSnXCbSYtHii?jPz!hKrbXCQra-_}3vbkD^kjE360VN9vdrKCg{yKbrUSv5&3}i0)=+*rz+&wmheQhBIR;+oLaLx~;Ar"
    ">-Mf}sBVY*m39p7p`+^de2MbrliY((xO3qZdCZU5w`6hHKr-S|BAb!oxXCOExgi3<<nV+YM}`-~lw=YG%sQP%9OY5vNWNT6Q7vt3"
    "d57+m1z<B;#hTMiw3o}kSGtG19E!unvc%@0-DZRx$v(13CXsp%7t9N@$;D9%=y6@0FXxx_JKvSR^(A;I{{vOi<IQ279#%>nPTK`6"
    "H?5_Tp@rx;pXr8@ZLXHMEN7D`WQ4Umfk1kbTBjngRr)b3Hdpx<S<3B)G=43Mja$+4un`qSTSJ{t!gWwL;R5M~FNhX=4K!uH<P@EW"
    "7xMq<F=~^|sp5OEgS|m(@NYCnuNI5&c)LR%)6MjFGT&Ba)50aPLEaJBc@Eu0E%wD&-LNnGG@a~Co)o^jr(qLYh<~~9KC;}%^VzKC"
    "37V(QoAvB2`p;Ei^UVP>i2V?E%u7`(yfy>P1NDr1|1gxG2hbvv1A3!(W<HxCZpb<!8;uT8{Ym+NOvf*94Aa(6)V*zU7zigwbavit"
    "SEt1r6@y$rS4B-0P5xo&^f?ya9SvR45Ve-?(mT+7bj}?V%XNC%8)n)JaMKioA3R&w$D7Efq7f?1w(#X{3_WIwkSN5ur{+AbEn<;b"
    "YOfw@U&xb2^W!$6nF<qFAA8Qf39U#>*Iqs{r_3_XLK0Ddu0>y6HnfH!_7YF@%WxJv%0KiO@DCb~ws8k}em8&@!jr6L^;r?N(R{UW"
    "!thW)KlPo(J=x#)!o9*^aoZ=te@ztplq?tD*-tzZZj)oEZ}>$j%a~{dKL`(KX*`riVbACy|3m&~5~JIummCc_><K@SP8KuldlD6%"
    "^H=<T%q?(v"
)
_PERM = np.frombuffer(
    zlib.decompress(base64.b85decode(_PERM_B85)), dtype="<u2"
).astype(np.int64)
_IDX = (_PERM[None, :] + _N * np.arange(_B)[:, None]).reshape(-1).astype(np.int32)

_NC = 2
_NS = 16
_NW = _NC * _NS
_PER_W = _ROWS // _NW
_CHUNK = 16
_NCHUNK = _PER_W // _CHUNK
_NBUF = 3
_NGROUP = _NCHUNK // _NBUF
_NTAIL = _NCHUNK - _NGROUP * _NBUF

@functools.cache
def _build_permute():
    mesh = plsc.VectorSubcoreMesh(core_axis_name="c", subcore_axis_name="s")

    @functools.partial(
        pl.kernel,
        mesh=mesh,
        out_type=jax.ShapeDtypeStruct((_ROWS, _D), jnp.float32),
        scratch_types=[
            pltpu.VMEM((_PER_W,), jnp.int32),
        ]
        + [pltpu.VMEM((_CHUNK, _D), jnp.float32)] * _NBUF
        + [pltpu.SemaphoreType.DMA] * (2 * _NBUF),
    )
    def _permute_rows(x_hbm, idx_hbm, out_hbm, idx_v, *scratch):
        bufs = scratch[:_NBUF]
        gsems = scratch[_NBUF : 2 * _NBUF]
        wsems = scratch[2 * _NBUF :]

        wid = lax.axis_index("s") * _NC + lax.axis_index("c")
        base = wid * _PER_W
        pltpu.sync_copy(idx_hbm.at[pl.ds(base, _PER_W)], idx_v)

        def gather_desc(k, b):
            src = x_hbm.at[idx_v.at[pl.ds(k * _CHUNK, _CHUNK)]]
            return pltpu.make_async_copy(src, bufs[b], gsems[b])

        def write_desc(k, b):
            dst = out_hbm.at[pl.ds(base + k * _CHUNK, _CHUNK)]
            return pltpu.make_async_copy(bufs[b], dst, wsems[b])

        for b in range(_NBUF):
            gather_desc(b, b).start()

        def body(g, carry):
            k0 = g * _NBUF
            for b in range(_NBUF):
                gather_desc(k0 + b, b).wait()
                write_desc(k0 + b, b).start()
            for b in range(_NBUF):
                k = k0 + b
                write_desc(k, b).wait()

                @pl.when(k + _NBUF < _NCHUNK)
                def _(b=b, k=k):
                    gather_desc(k + _NBUF, b).start()

            return carry

        lax.fori_loop(0, _NGROUP, body, 0)

        for t in range(_NTAIL):
            k = _NGROUP * _NBUF + t
            gather_desc(k, t).wait()
            write_desc(k, t).start()
        for t in range(_NTAIL):
            k = _NGROUP * _NBUF + t
            write_desc(k, t).wait()

    return _permute_rows


def kernel(x):
    xf = x.reshape(_ROWS, _D)
    out = _build_permute()(xf, jnp.asarray(_IDX))
    return out.reshape(_B, _N, _D)


# speedup vs baseline: 3.0632x; 1.0320x over previous
import base64
import functools
import zlib

import jax
import jax.numpy as jnp
import numpy as np
from jax import lax
from jax.experimental import pallas as pl
from jax.experimental.pallas import tpu as pltpu
from jax.experimental.pallas import tpu_sc as plsc

_B = 4
_N = 4096
_D = 2048
_ROWS = _B * _N

_PERM_B85 = (
    "c-jrr0|FR2768E5d8uvNwr$(C-TvCPZQHhO+qSzqhnU_WqhH`p@@V0^sR$*QVL#j#QW7Sp{sDMLlg8brmslg6JB$s<L_Y7(Ri9Qp"
    "!dp-(RWN)ExBLZLDO}ON@jEd|rxm4bS-%{g=E=iJR?2jjxA;VtS0|QhY&6^mr<2vm5!|1y)vM5+P=_Xj2yj~thoPuE+$VGCaXQV9"
    "bxZJOcF_s(6jHcZZiG%3s`)W?6#J_R%51)dUrx@d(lmkUBCf$AwIY1R?ZiE`*xtfQ<x^Qud$T}p@omfux`U12F@(e`)lOW5PFBlo"
    "3*X465SeXT@zzZ;CBrAZmW~Qr`3-VP4G7Ix%Fqyx6J2R>`qH)+XKjC6Q%O`4uaSLea(_%FgYx!>8zobto&1}>1iyF(yBAVG2Kxq="
    "6xGaajKe9{-R;8T^)lK~&2usA0Q5x0Mft@nvcM%F#I40|MOFOUrF8dMT6sa`f@yG0jKEWA9Tf>bhq0oPULy0+@;;wvKsTzE;Wsa?"
    "YM4#phdSz7Lp&HG#_}a7l1^$&7^`-nvveCdh%=Gj_KZhzy_WXB5Y;8OCCDJu9h!jGuV_@-S6@Uo%xGBx4$74J7!Z_`a+i<fHI-xs"
    "yq7fNFa2!&#2#a5P#IoS*EeO{c=1R?Q^jmjkq5=4kwqWAk9>kdP}an;(eXOD-n<}FO%~TuoiGkdw@VC_J^cc;&F?k6X^&7tyd;-d"
    "H1jOX$A?TNIU0{Jk69ewQd;`Sr4Q5}rGVp!(V+_(K&t@I*?1Efz=o6Vs2<OU44fvZF%3P)KA4B+*>kEtdy6}vkEpHN4r$0HU5RJm"
    "^L;&4kWUd)<j*kL{laTqV!njd_1DR8Q&T47`|Jla1~nH~d~UYR^fhNlOgLlT%LaNGxrjUQO01hDYOy@68rT)2uDOSo@>M*Me-bLI"
    "&U%1&gxW%PmIzvisO+B2t_a;mqr1!QzG@)m@$Gax8Utf}XE#{}eHOPgQS@bg6-uBkG7U}|UbDB1qY<LAh_9rIMzc!}F>wo4$|hx%"
    "T?IFs@0TsY16D)bU~5GZH;J~j75re8$|Czi7eWPCIh~yCz)M2@kXYqbR2DOb(G${~r{{mdw-ALVhJo$~AI^4?^(ZcRV~IJ1U$U*T"
    "B+jOD%M-LBzN-q6t#DYkW!J=VQ$`=4?`<pnDtx64{AN(Lb$CsG@)e>z>*pV<#?XzWwU2mA@zlnX!*LquDAVJ?cB#mwy7^3WyZOYn"
    "vJ2sw-_99og_y6xQ~6Y;5{xqu@LFSSX)+d9qql7Yx5QkRMMFK;i6s`(LkiMbE@6d3Trp3!rNz-<n_N~5O(74BA%~dM_Bww7M^Fm6"
    "TJ=L8<XgRgJ=1yY5VQ&p@LS1dUD+QBQ_L^5TkNz~NNL_qHn9uPOE~CaxWaTiFKpV-SpJ<(W=e|+D2}^_p6g<C932@BhAoiE3bIP3"
    "B$UNfd*l?i)Xt;H=`QnA#K8|`DcnkS(}i)N@Lsp^18G9n%Uo24vC^gVP0}x1foE`vUkX{o0eM%K@x%08lgQk0NpUwgCCZDAHXVD<"
    "ULYsehL~oYk4YQaeljQd$+RpV&fplZ%+=LRLQAqoU9??g8XI5Nmq%?TwcBmx{YgJlBs5_gV1kRJKH9ptJf~1aT;SKx6u@qm%4~|@"
    "`=O>E@20Big7b2AT!>A%D9e_jEn&F2&4<x{WKBqb2<y%-nLn_|#bbwkDx4oRV#VD)Uxr>s-&G^|$;OmXaT8KI{3IL1AJLR1W3A~S"
    "u4x5+7rsIbc^K}<{AwL9Mel`<I<tBMMeTL-7fy;!q=|W@uBylKgy^Z>;ZY_pKSz#>p{9w>VA_)V0=et%09hic%2~n7-DIa9i87O|"
    "zM4D1Ht|HH9ItG;t5pR1k|YuBU~(Z&&X@~6nQG~KlA?TxUuRqUg&z6MWSPjIQ<~Yd9_l0e*+;&o{7;O96}qqOM>Fb~XfDfvql%8`"
    "E{;sMh+(`I)bgj~5#1kh@M`L-j8EH;T4)8@!UAvSu7x8CLNb}fC|uBN&_}{Ymro{@J=h;U9Rif3{nZ6rIb=d3Q1ehkRTTY1UOG{1"
    "b|u*_I)oLn9r!(`$Q7Q`E#p^F5s^;yVHv}fFx7Wcb3_$$ja)Ok{3)~#y;D8$cfBBd#Xx@QyKX|5&vWrubXy3lrri))*njL8o{JyJ"
    "qNXgF#(wK9Hb0q&o0<0ZfxRKpqtxNAeGs0jdZwa@NWl#bZP<A|kbSX@#VC4%D%h$fn3epBh>w_@LAs%w{4Vjlhvz1j8(<!i+jK}c"
    "szE6}B0LW{<!<v(T@Bw{53w5FuybgR%P8JiOX7ysY@?ePTDfAfl5a|Gt4VN*J&>i{R5?K>aGm5t6A2chXy}rRFNcw~W{;2J3xij2"
    "fWRHU6h@P~Jhjm*g6#!q<S|uH6%&VD_i#qP<GKBI(woebzu*|1q&{1MhoL$Cl52%u=+-<1yYDKKYxKO2hn}Nrye~h2UnoHrt8TIl"
    "Q2ha4wJXdPlF=QqCEWsT$O>rZU-;}Ov!8B?^FiSw&dO(rO6nK->mQMb;w)-zrjjpou3b!$xss+U-cOr_+~~E)>kimu?43)B_KH5L"
    "HW8sPUvA^-gk*;)1ZB)Dyn}C)j`k9tHP$Zlw^@BP8=Yi=sAvj~V6<o^-=iQ_($%OQP`*vnVll~99Fx~zrNT9L%bizqs0ypuSiRFk"
    "^wj=Ql;)6gSY8$>WT#bR1vyRph8y0p3I3>vW((@AD5l?FR=Gpul^95}&?qpP?L_lMd-~h_Rt-p1xG6{B?Ru*Cs4v-Xc8_N2KCBnb"
    "ttK06W!=PIv!B%?yIqzL`Aip^P<=JM^)?tK`jFuBiY{!1mgFINjLw9PK8bD{KATP9w24m&t9Ics-fW)o4`e8=;`ZzPA)-v}ckwF}"
    ";c>2ztRH5(j3|c3VulGQzBx_TE5gB)rCaG5P2Cg}S*ODr{SG$~&Eloty2=~o!4cn540n#@M1y4mjL{LfBUED@!$VQk&Z0zL7BkQ="
    "yWY)4P85&{WMVdj8pmmcup5t7l_9Eq2$%2y-q$sj7fobe!W2*mjfEb3oUW)U)5u2S9<rY3qg$f&qzP~2>e2N+p6?*e@QnPnuHxI-"
    "Rf4nmXge!PxAC+(A`a?N*a6q{Yt+f7#C54)DRg2SjdkQwFLir(8GA@R!B2dB{E}voX$+R}^eyXccGl75gITN>Pp$LwbG(geE}yy>"
    "thB1nDQc{TspNv|NntMP?EB(ksE2v2>#@G<2#bqn$R4m)MT9bJowy}(+V3Ghz6d2zITsr)(0}5H8EO9M1!N&9%d@elJg@Ac*TWCn"
    "3pG%!d=0#hCPz<rLMi2B7uh!T`DHiW!5?B3a7~?!43y_ZJMvO)P*s#LhiEDJ%oIjv<aauoO_C$jbh%SKCb7gfNN3Bsb2Ku!N{*34"
    "a)xT)ve}ZlpGq&L+FEV_&qBHxi{FuvwyF6ll5mV0isCl9$ELm=<-YqHF1JAlsZ*Z&$v9t#LR({PKFZC$jjRciP;P$Ex8qfiK>zGz"
    "QwOf8uHu7U=Z5*lA-X7sk+>q?!V5Z0*T#?NCy|QR@vm)4_@G*uEh;YWtZJZXWGZ^;2D2A@k9`+@+1z-wedc2Es%)0UEDdjCv)h$="
    "m;8dJi?-+<w1f><=wYmCc&6UtOZ=%!?b7*-@SMK)dqi4P+E+D$(O;Zfq^1j1EIrwDkg-BVmc_?*rNMDb59x1s2kC>O+XCU7j$*3o"
    "TILq*3&lfITUsVCh5c?inf%9h>I*s&{pw?rSZWKcF7J?LDz?5$&%he~$DFriP&QH0XS73EV!KMMm)0MlUF~w3k#vIALPC3(Y2Kn3"
    "c&KmA#<1uzj@hANqYr#;_^SG&^dy`7CC2d@?u)<Y+v5E`BD?R?=#`KeHKDu182!fYve)V5@I>FEv!M#T&qwJ?y0F{BhUz5jpqh-D"
    "$_s9h9wsC5*Zhra$=BM~q%*Fl7qQ9g8&B*f+RijBStF|Gf_|(0M;}2|Cw&)uK<~r7RCQm`{;^A-peVw->I||Ox?xlLoS~IkVA|^;"
    "qz!6}zOtXLCQkq>S$jM)tac+sCtL@=f`;LVN*Lmi@wAiPB(w2yVvKo6|DemHf<75$@QVJGIfs|CSoSH(g@ww+c8V^c8RQbla8y;0"
    ")pkYTs4B#Z`h@xid8Est^nN<;q&k@h`b%hvj>_pOo++Vc^KX*kZmt$S4&MC-gruXf+&)=GuEP~zPAIDrxrKbSJTD)b!)6luBA(;B"
    "vLbrMPKQHono6!J;nOH<NJu+?r*%*sU7lGz+9pADcsbINwZJ1)Lw?z|)kWl4kwMJVNrJ(rU1FDm7B?%+=J3hPq_t!VR2ipKb=)yg"
    "JrtMPEz{Ce6K~}u(TN)0h3C_I?LTwRT({%dJT#Yo=85bg8;kwN-tx(Og>2!XppEPedhZl1E;58U=(XDo_swPXneC;g$OhERJrk8t"
    "0vt(9a-Go;5ydsfyV3dZK?r_GMH5YR2Yph_Cn<bQy3!9v5$M0r1kc2y(Qg0TKWBgGaaN4iwQ)!-J{O9zxUR041+8rbEAS1~CtR^j"
    "d0AC1+=2{rp-qSKu%+ZY%A)hRpZ+7MZL8q4Gy;i(Ci$agEhLcz{ClPSr?AkK#}~tUzL5^J(_~U@^lbCl@1S+qZZ*i4A?0Wn7l~~0"
    "=ODFcz*2=n;v1dj(wmF2u+4{)sQ3`yR;Tr33iLNPKY?Cj9zOXsY=Dz|hP#1#vL9>^sVWBI3?#QLDr(T1-l{}myANg<ZX#-;TbkiY"
    "uDGtCUh*JE+2=MDr1j<9MKQ-F&<YI;Z$sCR-K{n~a0InkeC40iNz{e5CLcmcn}Fujk$DNZk97!v^l(dYc5(-XnN<2^NX;9<ZrhMf"
    "cY__%(cEn)OdiuHwhtWlHAz{UIdlul%zj?O{-%9Jj1W1@RI7Csi%5UIQ(j<Kag7kkjbl?7<2lU-zn`@*t<@ltBXl7j99N^@imFJj"
    "U}R209(mK|B?DwueP3p#Uv*+i)kuGlWj6<PIh7EvqvPxgcbNV5MP1ae*5_k`aT$ET&ksvk19Vq>;&<%}`=9+`_JG#M{7RJ0B;X4}"
    "Y_S(#!|muc)X+~<d(8VV-zA6n@(;epvD)Ao+Z8M?3{^>NZaoLa@TI1g-9xYH#XL31rBCBxc(R|4cCvZ;9o#|_NES93l#e5;^1fuA"
    "KftPy+@zOZjI%*pxd|mFa~%`;&3I_2m(tPvKYT#_04hrBLM)fPsWY=ha-h12Ys>7Okug3Kdxna**RVSLryk<VXudjV^6PZupiSUE"
    "Kw5TKPA5lfS9b#%n(bnUo`E0Ro}w$;6>5;y2HP{V66r5K;U%^P^waf1afl)&vXlBCX^hv(Pb9X#9%AYU{yyH#6Oco=6(4IE{F5C}"
    "+TcS<d(q!^8s#w!!W7?Doinf1JT}*4l<`A9R@-b=8)zZb+7)wk$=+bma0}3ZWW{BmzaA+asY2eW6RMrAU=pE%Xgm7NzM)Y(y(%x`"
    "&@|y5DQ#xyVx%|QL;lL_qK7J>ir59Xq3Wf7`5bJX$SU8Pf9|b~XRDi&c&kYxVyeihv)pKx!x)kfB{2&?xL542?#WNf5_X6@8v+gC"
    "c&HgF^RZ;SeC{@xb)llX8K%)(v<;p}%A0b~8;=P`*-EkkUuE-kM3)n4^Gmd+iG&jAH@q`Dqq4wyyNfjy=WK0%kaZ(U<hO^#E%t&n"
    "Bk#p%e@_09PgO;iheY!u>0id&A2QEmcID(ZH3#R!x6~V&C^)FEtD}wNB7aNo;5xP%S|rBE5wIV9CmPDDxqg6-%O03EdN5rS`iPJ4"
    "UFIMQM5gc=zPK`?0E(eLuyG27@@Oah&OiFHkcyqBt7vDESs%dbAOcS%o|{EvFX~PQ*v#^<8m+h4jiQek82-49GzA~(4$~9R-2dkX"
    "iYPJ(ob}D@AX?eB)VVxkCw(OL0EUTrwjHbrxm0gIhnM0<@pZjhPuFMRvUl!}zJ%KQ33e|(BMR#s;tq=iEy#60mi87Y@MrQ(9Hf8A"
    "A)lDc4$;FR{=t?}>*)@^!={$sZ81nj54eKlrMMA}lbR;2tDy(UOFmD?!$;GqG?{<EABe0h4j<;cIANcQS7;y_tc#Q0va2euuevuz"
    "(n=_c8i!B&QK%wcX!58@Jhn}vKG4W+i>@nYnIfnoG(pQ{J-@`wRuknSh#5Ykk+_a*L?(zurZ|g*$B~K9N@Ne^Of~#Qx5Lr&c3hW_"
    "HI>A7F~@YHEo8Fr{|;$v9-sB358MdXm__F!{7ipbp5QU~C)&ps(`!{Nty}_oQ|)yN*$TTECnbYnw$0&Q;reQ$uFaN))B2JsVsdzn"
    "W3t+68jnhrq3Ak~D+(L<YLbiWS9@G?d{eZfBi(aS-~X0z{Wx@1kCRQq5PT>6VvC`jX~d_x8a%DZN>9N94I(-J$5Xioq5$s<XJu|k"
    "u9K>LhM9j*R2_xYyt!T~U)eu22I?zYqk?WK`=?IYU%H1~rtavLI5L{Ss@RWwB-~~hSv|a2eh#PTS~7wa5;??2brfY4@8|_tmo3%_"
    "MP2mRt@B;UZg!r$L%mfbe+Ol7FHH^=pEpM*X|+&W6JHk=(q(c#PC|=?{P0-3_D6kma}XU1pT!tABMjG5b$*tP7jW<7?U2=$lx1-f"
    "H48n!XI%>_V3mL1--zl^$^5`CL~7Q+<na;YGB|`T(#JS1WKbJnh#h28q8@ZHb2gR<IF9HPnqnJ*jl}D_sAhe@yhPXu#eEy`9nXMu"
    "=r`%@B9aO4l$GQS(HVA~6~|WG*Ns&vHe61pLwGB;92J$%$Szt)KXjFBdOHBG_H)S}-xE@@xjLh&4L?zNQip8ATSZdW4pr4pXf@a-"
    "H`8l!7kQ1_glVX($jKIq=4!5(Pdnl(F1@>DT3I7w&}_J)=;{|jUUkN;GE4OrJ<tzQjp<L@T2IAS=?1^cU4~?~p029elLv^{MwZKn"
    "I)^!q-?#;K5Ff}(sS>`CTEh2}+v01;j{89~T%OLB+d~WW!#5;-RYj83RmYvfKl~qxr&dtKR?`ZsrhAEg(srtSm~1EOg0`)ErT4p@"
    "_AYD8hXCRc(Q&m2zmiGtJsq2EaVPX=H-~ugQQY9SNPN5p&4)GaiCd2csP2A^N@}m;L85e6p_2Oe;tTs7x|n~yAD@Kl3m#5`G8s@K"
    "dMmsLDd}GJmoE?5Tw<LHr4tFlJ64@tHB;<hlLLWYs?MNTvZY%Y2EcQfN;*YkB>6;C<4-k)KD4k{iJw4T9~pPY+g&f3Q)FPzWEz&+"
    "jnqd75x>HB*aa0xe{)M_hu^}`6X-0PW=r5w#F(KX6*=w_v3Mc4#;&GFZoZOLFw1vvMiiu(a8?vY=ck~%lkvC|8bJm@T3GD1;AFfR"
    "okytoE55O_rZX?88nP>90C{ZE%R<zs0%9WnVAk_HHj`^i^5A9RE6PaY;pP6GSmCy*d^{<S#^dpAYz-NNi<0TSKH6`hgvS1+tLs`o"
    "SF=zgceUs&`N+i%bCt*Cd<u)n7nNO~B@xwZ|5>I8B&3uP!akA0j8YyI@sS~kpCs;v!{nuY=6kxsVG7USe}oBmiOv<C;>V#L=?cBr"
    "Q##FLB}H*Hxlm^`fbQ^eEH6$*vV^hnHX0#MgrBkl+aas^eQGe@rkj&><S**$^CE3!NI@HklWw%BqZ`OHbhDkn&*RrJp&g`~`DWrM"
    "87;P$Chn0+4Gmp-)<V~^pHNj&-Hi{^*_4n~rNuvCFj+*W;o3AQEsdVpHhjB1=I-0r<{~*2F2E}L*IlNu(0{a1;5MFH<4=+$<UFfS"
    "W2@D;y%MrjSScDC;eV=`<dtvgKGL43x$DidLIH9GJu!XIc-7n7W1C>Eyovg`IWirzg57c?sYOQ6p1L-Q7=}1!s-jL}zs(Fw{TE(L"
    "4TZAuKqz3xqM4$*?k0-)N#sBBO&{a^Ft9GLRsSbb(VcD-DWb27&L%S37z&v$IFo<nKn|yOeJ+zp?{if`Q+1Pe@s(&@^&Ksgot&|S"
    "WiBb%&X71{GLh^FnTT(ZcT{(~Uvw8Y&|W`;P7M!r0m^JU7DG>_>)0>znbwC#><c)ZUO!~f2%|%3eqWfaK|O<#b5v^ehO9BxY;|Qq"
    "Lpct<kq9UuOXsf2JkXI{hx%kAzM*fjWI8(Si%!|8JT1SdqoYP9q3`1cx&Csth-(JwXwZRH5Gln3*$5)qj^qf;3#Zv)+5ir~1T+jz"
    "=^?%VPRYNSnXCbSYtHii?jPz!hKrbXCQra-_}3vbkD^kjE360VN9vdrKCg{yKbrUSv5&3}i0)=+*rz+&wmheQhBIR;+oLaLx~;Ar"
    ">-Mf}sBVY*m39p7p`+^de2MbrliY((xO3qZdCZU5w`6hHKr-S|BAb!oxXCOExgi3<<nV+YM}`-~lw=YG%sQP%9OY5vNWNT6Q7vt3"
    "d57+m1z<B;#hTMiw3o}kSGtG19E!unvc%@0-DZRx$v(13CXsp%7t9N@$;D9%=y6@0FXxx_JKvSR^(A;I{{vOi<IQ279#%>nPTK`6"
    "H?5_Tp@rx;pXr8@ZLXHMEN7D`WQ4Umfk1kbTBjngRr)b3Hdpx<S<3B)G=43Mja$+4un`qSTSJ{t!gWwL;R5M~FNhX=4K!uH<P@EW"
    "7xMq<F=~^|sp5OEgS|m(@NYCnuNI5&c)LR%)6MjFGT&Ba)50aPLEaJBc@Eu0E%wD&-LNnGG@a~Co)o^jr(qLYh<~~9KC;}%^VzKC"
    "37V(QoAvB2`p;Ei^UVP>i2V?E%u7`(yfy>P1NDr1|1gxG2hbvv1A3!(W<HxCZpb<!8;uT8{Ym+NOvf*94Aa(6)V*zU7zigwbavit"
    "SEt1r6@y$rS4B-0P5xo&^f?ya9SvR45Ve-?(mT+7bj}?V%XNC%8)n)JaMKioA3R&w$D7Efq7f?1w(#X{3_WIwkSN5ur{+AbEn<;b"
    "YOfw@U&xb2^W!$6nF<qFAA8Qf39U#>*Iqs{r_3_XLK0Ddu0>y6HnfH!_7YF@%WxJv%0KiO@DCb~ws8k}em8&@!jr6L^;r?N(R{UW"
    "!thW)KlPo(J=x#)!o9*^aoZ=te@ztplq?tD*-tzZZj)oEZ}>$j%a~{dKL`(KX*`riVbACy|3m&~5~JIummCc_><K@SP8KuldlD6%"
    "^H=<T%q?(v"
)
_PERM = np.frombuffer(
    zlib.decompress(base64.b85decode(_PERM_B85)), dtype="<u2"
).astype(np.int64)
_IDX = (_PERM[None, :] + _N * np.arange(_B)[:, None]).reshape(-1).astype(np.int32)

_NC = 2
_NS = 16
_NW = _NC * _NS
_PER_W = _ROWS // _NW
_CHUNK = 8
_NCHUNK = _PER_W // _CHUNK
_NBUF = 4
_NGROUP = _NCHUNK // _NBUF
_NTAIL = _NCHUNK - _NGROUP * _NBUF

@functools.cache
def _build_permute():
    mesh = plsc.VectorSubcoreMesh(core_axis_name="c", subcore_axis_name="s")

    @functools.partial(
        pl.kernel,
        mesh=mesh,
        out_type=jax.ShapeDtypeStruct((_ROWS, _D), jnp.float32),
        scratch_types=[
            pltpu.VMEM((_PER_W,), jnp.int32),
        ]
        + [pltpu.VMEM((_CHUNK, _D), jnp.float32)] * _NBUF
        + [pltpu.SemaphoreType.DMA] * (2 * _NBUF),
    )
    def _permute_rows(x_hbm, idx_hbm, out_hbm, idx_v, *scratch):
        bufs = scratch[:_NBUF]
        gsems = scratch[_NBUF : 2 * _NBUF]
        wsems = scratch[2 * _NBUF :]

        wid = lax.axis_index("s") * _NC + lax.axis_index("c")
        base = wid * _PER_W
        pltpu.sync_copy(idx_hbm.at[pl.ds(base, _PER_W)], idx_v)

        def gather_desc(k, b):
            src = x_hbm.at[idx_v.at[pl.ds(k * _CHUNK, _CHUNK)]]
            return pltpu.make_async_copy(src, bufs[b], gsems[b])

        def write_desc(k, b):
            dst = out_hbm.at[pl.ds(base + k * _CHUNK, _CHUNK)]
            return pltpu.make_async_copy(bufs[b], dst, wsems[b])

        for b in range(_NBUF):
            gather_desc(b, b).start()

        def body(g, carry):
            k0 = g * _NBUF
            for b in range(_NBUF):
                gather_desc(k0 + b, b).wait()
                write_desc(k0 + b, b).start()
            for b in range(_NBUF):
                k = k0 + b
                write_desc(k, b).wait()

                @pl.when(k + _NBUF < _NCHUNK)
                def _(b=b, k=k):
                    gather_desc(k + _NBUF, b).start()

            return carry

        lax.fori_loop(0, _NGROUP, body, 0)

        for t in range(_NTAIL):
            k = _NGROUP * _NBUF + t
            gather_desc(k, t).wait()
            write_desc(k, t).start()
        for t in range(_NTAIL):
            k = _NGROUP * _NBUF + t
            write_desc(k, t).wait()

    return _permute_rows


def kernel(x):
    xf = x.reshape(_ROWS, _D)
    out = _build_permute()(xf, jnp.asarray(_IDX))
    return out.reshape(_B, _N, _D)


# speedup vs baseline: 3.0849x; 1.0071x over previous
import base64
import functools
import zlib

import jax
import jax.numpy as jnp
import numpy as np
from jax import lax
from jax.experimental import pallas as pl
from jax.experimental.pallas import tpu as pltpu
from jax.experimental.pallas import tpu_sc as plsc

_B = 4
_N = 4096
_D = 2048
_ROWS = _B * _N

_PERM_B85 = (
    "c-jrr0|FR2768E5d8uvNwr$(C-TvCPZQHhO+qSzqhnU_WqhH`p@@V0^sR$*QVL#j#QW7Sp{sDMLlg8brmslg6JB$s<L_Y7(Ri9Qp"
    "!dp-(RWN)ExBLZLDO}ON@jEd|rxm4bS-%{g=E=iJR?2jjxA;VtS0|QhY&6^mr<2vm5!|1y)vM5+P=_Xj2yj~thoPuE+$VGCaXQV9"
    "bxZJOcF_s(6jHcZZiG%3s`)W?6#J_R%51)dUrx@d(lmkUBCf$AwIY1R?ZiE`*xtfQ<x^Qud$T}p@omfux`U12F@(e`)lOW5PFBlo"
    "3*X465SeXT@zzZ;CBrAZmW~Qr`3-VP4G7Ix%Fqyx6J2R>`qH)+XKjC6Q%O`4uaSLea(_%FgYx!>8zobto&1}>1iyF(yBAVG2Kxq="
    "6xGaajKe9{-R;8T^)lK~&2usA0Q5x0Mft@nvcM%F#I40|MOFOUrF8dMT6sa`f@yG0jKEWA9Tf>bhq0oPULy0+@;;wvKsTzE;Wsa?"
    "YM4#phdSz7Lp&HG#_}a7l1^$&7^`-nvveCdh%=Gj_KZhzy_WXB5Y;8OCCDJu9h!jGuV_@-S6@Uo%xGBx4$74J7!Z_`a+i<fHI-xs"
    "yq7fNFa2!&#2#a5P#IoS*EeO{c=1R?Q^jmjkq5=4kwqWAk9>kdP}an;(eXOD-n<}FO%~TuoiGkdw@VC_J^cc;&F?k6X^&7tyd;-d"
    "H1jOX$A?TNIU0{Jk69ewQd;`Sr4Q5}rGVp!(V+_(K&t@I*?1Efz=o6Vs2<OU44fvZF%3P)KA4B+*>kEtdy6}vkEpHN4r$0HU5RJm"
    "^L;&4kWUd)<j*kL{laTqV!njd_1DR8Q&T47`|Jla1~nH~d~UYR^fhNlOgLlT%LaNGxrjUQO01hDYOy@68rT)2uDOSo@>M*Me-bLI"
    "&U%1&gxW%PmIzvisO+B2t_a;mqr1!QzG@)m@$Gax8Utf}XE#{}eHOPgQS@bg6-uBkG7U}|UbDB1qY<LAh_9rIMzc!}F>wo4$|hx%"
    "T?IFs@0TsY16D)bU~5GZH;J~j75re8$|Czi7eWPCIh~yCz)M2@kXYqbR2DOb(G${~r{{mdw-ALVhJo$~AI^4?^(ZcRV~IJ1U$U*T"
    "B+jOD%M-LBzN-q6t#DYkW!J=VQ$`=4?`<pnDtx64{AN(Lb$CsG@)e>z>*pV<#?XzWwU2mA@zlnX!*LquDAVJ?cB#mwy7^3WyZOYn"
    "vJ2sw-_99og_y6xQ~6Y;5{xqu@LFSSX)+d9qql7Yx5QkRMMFK;i6s`(LkiMbE@6d3Trp3!rNz-<n_N~5O(74BA%~dM_Bww7M^Fm6"
    "TJ=L8<XgRgJ=1yY5VQ&p@LS1dUD+QBQ_L^5TkNz~NNL_qHn9uPOE~CaxWaTiFKpV-SpJ<(W=e|+D2}^_p6g<C932@BhAoiE3bIP3"
    "B$UNfd*l?i)Xt;H=`QnA#K8|`DcnkS(}i)N@Lsp^18G9n%Uo24vC^gVP0}x1foE`vUkX{o0eM%K@x%08lgQk0NpUwgCCZDAHXVD<"
    "ULYsehL~oYk4YQaeljQd$+RpV&fplZ%+=LRLQAqoU9??g8XI5Nmq%?TwcBmx{YgJlBs5_gV1kRJKH9ptJf~1aT;SKx6u@qm%4~|@"
    "`=O>E@20Big7b2AT!>A%D9e_jEn&F2&4<x{WKBqb2<y%-nLn_|#bbwkDx4oRV#VD)Uxr>s-&G^|$;OmXaT8KI{3IL1AJLR1W3A~S"
    "u4x5+7rsIbc^K}<{AwL9Mel`<I<tBMMeTL-7fy;!q=|W@uBylKgy^Z>;ZY_pKSz#>p{9w>VA_)V0=et%09hic%2~n7-DIa9i87O|"
    "zM4D1Ht|HH9ItG;t5pR1k|YuBU~(Z&&X@~6nQG~KlA?TxUuRqUg&z6MWSPjIQ<~Yd9_l0e*+;&o{7;O96}qqOM>Fb~XfDfvql%8`"
    "E{;sMh+(`I)bgj~5#1kh@M`L-j8EH;T4)8@!UAvSu7x8CLNb}fC|uBN&_}{Ymro{@J=h;U9Rif3{nZ6rIb=d3Q1ehkRTTY1UOG{1"
    "b|u*_I)oLn9r!(`$Q7Q`E#p^F5s^;yVHv}fFx7Wcb3_$$ja)Ok{3)~#y;D8$cfBBd#Xx@QyKX|5&vWrubXy3lrri))*njL8o{JyJ"
    "qNXgF#(wK9Hb0q&o0<0ZfxRKpqtxNAeGs0jdZwa@NWl#bZP<A|kbSX@#VC4%D%h$fn3epBh>w_@LAs%w{4Vjlhvz1j8(<!i+jK}c"
    "szE6}B0LW{<!<v(T@Bw{53w5FuybgR%P8JiOX7ysY@?ePTDfAfl5a|Gt4VN*J&>i{R5?K>aGm5t6A2chXy}rRFNcw~W{;2J3xij2"
    "fWRHU6h@P~Jhjm*g6#!q<S|uH6%&VD_i#qP<GKBI(woebzu*|1q&{1MhoL$Cl52%u=+-<1yYDKKYxKO2hn}Nrye~h2UnoHrt8TIl"
    "Q2ha4wJXdPlF=QqCEWsT$O>rZU-;}Ov!8B?^FiSw&dO(rO6nK->mQMb;w)-zrjjpou3b!$xss+U-cOr_+~~E)>kimu?43)B_KH5L"
    "HW8sPUvA^-gk*;)1ZB)Dyn}C)j`k9tHP$Zlw^@BP8=Yi=sAvj~V6<o^-=iQ_($%OQP`*vnVll~99Fx~zrNT9L%bizqs0ypuSiRFk"
    "^wj=Ql;)6gSY8$>WT#bR1vyRph8y0p3I3>vW((@AD5l?FR=Gpul^95}&?qpP?L_lMd-~h_Rt-p1xG6{B?Ru*Cs4v-Xc8_N2KCBnb"
    "ttK06W!=PIv!B%?yIqzL`Aip^P<=JM^)?tK`jFuBiY{!1mgFINjLw9PK8bD{KATP9w24m&t9Ics-fW)o4`e8=;`ZzPA)-v}ckwF}"
    ";c>2ztRH5(j3|c3VulGQzBx_TE5gB)rCaG5P2Cg}S*ODr{SG$~&Eloty2=~o!4cn540n#@M1y4mjL{LfBUED@!$VQk&Z0zL7BkQ="
    "yWY)4P85&{WMVdj8pmmcup5t7l_9Eq2$%2y-q$sj7fobe!W2*mjfEb3oUW)U)5u2S9<rY3qg$f&qzP~2>e2N+p6?*e@QnPnuHxI-"
    "Rf4nmXge!PxAC+(A`a?N*a6q{Yt+f7#C54)DRg2SjdkQwFLir(8GA@R!B2dB{E}voX$+R}^eyXccGl75gITN>Pp$LwbG(geE}yy>"
    "thB1nDQc{TspNv|NntMP?EB(ksE2v2>#@G<2#bqn$R4m)MT9bJowy}(+V3Ghz6d2zITsr)(0}5H8EO9M1!N&9%d@elJg@Ac*TWCn"
    "3pG%!d=0#hCPz<rLMi2B7uh!T`DHiW!5?B3a7~?!43y_ZJMvO)P*s#LhiEDJ%oIjv<aauoO_C$jbh%SKCb7gfNN3Bsb2Ku!N{*34"
    "a)xT)ve}ZlpGq&L+FEV_&qBHxi{FuvwyF6ll5mV0isCl9$ELm=<-YqHF1JAlsZ*Z&$v9t#LR({PKFZC$jjRciP;P$Ex8qfiK>zGz"
    "QwOf8uHu7U=Z5*lA-X7sk+>q?!V5Z0*T#?NCy|QR@vm)4_@G*uEh;YWtZJZXWGZ^;2D2A@k9`+@+1z-wedc2Es%)0UEDdjCv)h$="
    "m;8dJi?-+<w1f><=wYmCc&6UtOZ=%!?b7*-@SMK)dqi4P+E+D$(O;Zfq^1j1EIrwDkg-BVmc_?*rNMDb59x1s2kC>O+XCU7j$*3o"
    "TILq*3&lfITUsVCh5c?inf%9h>I*s&{pw?rSZWKcF7J?LDz?5$&%he~$DFriP&QH0XS73EV!KMMm)0MlUF~w3k#vIALPC3(Y2Kn3"
    "c&KmA#<1uzj@hANqYr#;_^SG&^dy`7CC2d@?u)<Y+v5E`BD?R?=#`KeHKDu182!fYve)V5@I>FEv!M#T&qwJ?y0F{BhUz5jpqh-D"
    "$_s9h9wsC5*Zhra$=BM~q%*Fl7qQ9g8&B*f+RijBStF|Gf_|(0M;}2|Cw&)uK<~r7RCQm`{;^A-peVw->I||Ox?xlLoS~IkVA|^;"
    "qz!6}zOtXLCQkq>S$jM)tac+sCtL@=f`;LVN*Lmi@wAiPB(w2yVvKo6|DemHf<75$@QVJGIfs|CSoSH(g@ww+c8V^c8RQbla8y;0"
    ")pkYTs4B#Z`h@xid8Est^nN<;q&k@h`b%hvj>_pOo++Vc^KX*kZmt$S4&MC-gruXf+&)=GuEP~zPAIDrxrKbSJTD)b!)6luBA(;B"
    "vLbrMPKQHono6!J;nOH<NJu+?r*%*sU7lGz+9pADcsbINwZJ1)Lw?z|)kWl4kwMJVNrJ(rU1FDm7B?%+=J3hPq_t!VR2ipKb=)yg"
    "JrtMPEz{Ce6K~}u(TN)0h3C_I?LTwRT({%dJT#Yo=85bg8;kwN-tx(Og>2!XppEPedhZl1E;58U=(XDo_swPXneC;g$OhERJrk8t"
    "0vt(9a-Go;5ydsfyV3dZK?r_GMH5YR2Yph_Cn<bQy3!9v5$M0r1kc2y(Qg0TKWBgGaaN4iwQ)!-J{O9zxUR041+8rbEAS1~CtR^j"
    "d0AC1+=2{rp-qSKu%+ZY%A)hRpZ+7MZL8q4Gy;i(Ci$agEhLcz{ClPSr?AkK#}~tUzL5^J(_~U@^lbCl@1S+qZZ*i4A?0Wn7l~~0"
    "=ODFcz*2=n;v1dj(wmF2u+4{)sQ3`yR;Tr33iLNPKY?Cj9zOXsY=Dz|hP#1#vL9>^sVWBI3?#QLDr(T1-l{}myANg<ZX#-;TbkiY"
    "uDGtCUh*JE+2=MDr1j<9MKQ-F&<YI;Z$sCR-K{n~a0InkeC40iNz{e5CLcmcn}Fujk$DNZk97!v^l(dYc5(-XnN<2^NX;9<ZrhMf"
    "cY__%(cEn)OdiuHwhtWlHAz{UIdlul%zj?O{-%9Jj1W1@RI7Csi%5UIQ(j<Kag7kkjbl?7<2lU-zn`@*t<@ltBXl7j99N^@imFJj"
    "U}R209(mK|B?DwueP3p#Uv*+i)kuGlWj6<PIh7EvqvPxgcbNV5MP1ae*5_k`aT$ET&ksvk19Vq>;&<%}`=9+`_JG#M{7RJ0B;X4}"
    "Y_S(#!|muc)X+~<d(8VV-zA6n@(;epvD)Ao+Z8M?3{^>NZaoLa@TI1g-9xYH#XL31rBCBxc(R|4cCvZ;9o#|_NES93l#e5;^1fuA"
    "KftPy+@zOZjI%*pxd|mFa~%`;&3I_2m(tPvKYT#_04hrBLM)fPsWY=ha-h12Ys>7Okug3Kdxna**RVSLryk<VXudjV^6PZupiSUE"
    "Kw5TKPA5lfS9b#%n(bnUo`E0Ro}w$;6>5;y2HP{V66r5K;U%^P^waf1afl)&vXlBCX^hv(Pb9X#9%AYU{yyH#6Oco=6(4IE{F5C}"
    "+TcS<d(q!^8s#w!!W7?Doinf1JT}*4l<`A9R@-b=8)zZb+7)wk$=+bma0}3ZWW{BmzaA+asY2eW6RMrAU=pE%Xgm7NzM)Y(y(%x`"
    "&@|y5DQ#xyVx%|QL;lL_qK7J>ir59Xq3Wf7`5bJX$SU8Pf9|b~XRDi&c&kYxVyeihv)pKx!x)kfB{2&?xL542?#WNf5_X6@8v+gC"
    "c&HgF^RZ;SeC{@xb)llX8K%)(v<;p}%A0b~8;=P`*-EkkUuE-kM3)n4^Gmd+iG&jAH@q`Dqq4wyyNfjy=WK0%kaZ(U<hO^#E%t&n"
    "Bk#p%e@_09PgO;iheY!u>0id&A2QEmcID(ZH3#R!x6~V&C^)FEtD}wNB7aNo;5xP%S|rBE5wIV9CmPDDxqg6-%O03EdN5rS`iPJ4"
    "UFIMQM5gc=zPK`?0E(eLuyG27@@Oah&OiFHkcyqBt7vDESs%dbAOcS%o|{EvFX~PQ*v#^<8m+h4jiQek82-49GzA~(4$~9R-2dkX"
    "iYPJ(ob}D@AX?eB)VVxkCw(OL0EUTrwjHbrxm0gIhnM0<@pZjhPuFMRvUl!}zJ%KQ33e|(BMR#s;tq=iEy#60mi87Y@MrQ(9Hf8A"
    "A)lDc4$;FR{=t?}>*)@^!={$sZ81nj54eKlrMMA}lbR;2tDy(UOFmD?!$;GqG?{<EABe0h4j<;cIANcQS7;y_tc#Q0va2euuevuz"
    "(n=_c8i!B&QK%wcX!58@Jhn}vKG4W+i>@nYnIfnoG(pQ{J-@`wRuknSh#5Ykk+_a*L?(zurZ|g*$B~K9N@Ne^Of~#Qx5Lr&c3hW_"
    "HI>A7F~@YHEo8Fr{|;$v9-sB358MdXm__F!{7ipbp5QU~C)&ps(`!{Nty}_oQ|)yN*$TTECnbYnw$0&Q;reQ$uFaN))B2JsVsdzn"
    "W3t+68jnhrq3Ak~D+(L<YLbiWS9@G?d{eZfBi(aS-~X0z{Wx@1kCRQq5PT>6VvC`jX~d_x8a%DZN>9N94I(-J$5Xioq5$s<XJu|k"
    "u9K>LhM9j*R2_xYyt!T~U)eu22I?zYqk?WK`=?IYU%H1~rtavLI5L{Ss@RWwB-~~hSv|a2eh#PTS~7wa5;??2brfY4@8|_tmo3%_"
    "MP2mRt@B;UZg!r$L%mfbe+Ol7FHH^=pEpM*X|+&W6JHk=(q(c#PC|=?{P0-3_D6kma}XU1pT!tABMjG5b$*tP7jW<7?U2=$lx1-f"
    "H48n!XI%>_V3mL1--zl^$^5`CL~7Q+<na;YGB|`T(#JS1WKbJnh#h28q8@ZHb2gR<IF9HPnqnJ*jl}D_sAhe@yhPXu#eEy`9nXMu"
    "=r`%@B9aO4l$GQS(HVA~6~|WG*Ns&vHe61pLwGB;92J$%$Szt)KXjFBdOHBG_H)S}-xE@@xjLh&4L?zNQip8ATSZdW4pr4pXf@a-"
    "H`8l!7kQ1_glVX($jKIq=4!5(Pdnl(F1@>DT3I7w&}_J)=;{|jUUkN;GE4OrJ<tzQjp<L@T2IAS=?1^cU4~?~p029elLv^{MwZKn"
    "I)^!q-?#;K5Ff}(sS>`CTEh2}+v01;j{89~T%OLB+d~WW!#5;-RYj83RmYvfKl~qxr&dtKR?`ZsrhAEg(srtSm~1EOg0`)ErT4p@"
    "_AYD8hXCRc(Q&m2zmiGtJsq2EaVPX=H-~ugQQY9SNPN5p&4)GaiCd2csP2A^N@}m;L85e6p_2Oe;tTs7x|n~yAD@Kl3m#5`G8s@K"
    "dMmsLDd}GJmoE?5Tw<LHr4tFlJ64@tHB;<hlLLWYs?MNTvZY%Y2EcQfN;*YkB>6;C<4-k)KD4k{iJw4T9~pPY+g&f3Q)FPzWEz&+"
    "jnqd75x>HB*aa0xe{)M_hu^}`6X-0PW=r5w#F(KX6*=w_v3Mc4#;&GFZoZOLFw1vvMiiu(a8?vY=ck~%lkvC|8bJm@T3GD1;AFfR"
    "okytoE55O_rZX?88nP>90C{ZE%R<zs0%9WnVAk_HHj`^i^5A9RE6PaY;pP6GSmCy*d^{<S#^dpAYz-NNi<0TSKH6`hgvS1+tLs`o"
    "SF=zgceUs&`N+i%bCt*Cd<u)n7nNO~B@xwZ|5>I8B&3uP!akA0j8YyI@sS~kpCs;v!{nuY=6kxsVG7USe}oBmiOv<C;>V#L=?cBr"
    "Q##FLB}H*Hxlm^`fbQ^eEH6$*vV^hnHX0#MgrBkl+aas^eQGe@rkj&><S**$^CE3!NI@HklWw%BqZ`OHbhDkn&*RrJp&g`~`DWrM"
    "87;P$Chn0+4Gmp-)<V~^pHNj&-Hi{^*_4n~rNuvCFj+*W;o3AQEsdVpHhjB1=I-0r<{~*2F2E}L*IlNu(0{a1;5MFH<4=+$<UFfS"
    "W2@D;y%MrjSScDC;eV=`<dtvgKGL43x$DidLIH9GJu!XIc-7n7W1C>Eyovg`IWirzg57c?sYOQ6p1L-Q7=}1!s-jL}zs(Fw{TE(L"
    "4TZAuKqz3xqM4$*?k0-)N#sBBO&{a^Ft9GLRsSbb(VcD-DWb27&L%S37z&v$IFo<nKn|yOeJ+zp?{if`Q+1Pe@s(&@^&Ksgot&|S"
    "WiBb%&X71{GLh^FnTT(ZcT{(~Uvw8Y&|W`;P7M!r0m^JU7DG>_>)0>znbwC#><c)ZUO!~f2%|%3eqWfaK|O<#b5v^ehO9BxY;|Qq"
    "Lpct<kq9UuOXsf2JkXI{hx%kAzM*fjWI8(Si%!|8JT1SdqoYP9q3`1cx&Csth-(JwXwZRH5Gln3*$5)qj^qf;3#Zv)+5ir~1T+jz"
    "=^?%VPRYNSnXCbSYtHii?jPz!hKrbXCQra-_}3vbkD^kjE360VN9vdrKCg{yKbrUSv5&3}i0)=+*rz+&wmheQhBIR;+oLaLx~;Ar"
    ">-Mf}sBVY*m39p7p`+^de2MbrliY((xO3qZdCZU5w`6hHKr-S|BAb!oxXCOExgi3<<nV+YM}`-~lw=YG%sQP%9OY5vNWNT6Q7vt3"
    "d57+m1z<B;#hTMiw3o}kSGtG19E!unvc%@0-DZRx$v(13CXsp%7t9N@$;D9%=y6@0FXxx_JKvSR^(A;I{{vOi<IQ279#%>nPTK`6"
    "H?5_Tp@rx;pXr8@ZLXHMEN7D`WQ4Umfk1kbTBjngRr)b3Hdpx<S<3B)G=43Mja$+4un`qSTSJ{t!gWwL;R5M~FNhX=4K!uH<P@EW"
    "7xMq<F=~^|sp5OEgS|m(@NYCnuNI5&c)LR%)6MjFGT&Ba)50aPLEaJBc@Eu0E%wD&-LNnGG@a~Co)o^jr(qLYh<~~9KC;}%^VzKC"
    "37V(QoAvB2`p;Ei^UVP>i2V?E%u7`(yfy>P1NDr1|1gxG2hbvv1A3!(W<HxCZpb<!8;uT8{Ym+NOvf*94Aa(6)V*zU7zigwbavit"
    "SEt1r6@y$rS4B-0P5xo&^f?ya9SvR45Ve-?(mT+7bj}?V%XNC%8)n)JaMKioA3R&w$D7Efq7f?1w(#X{3_WIwkSN5ur{+AbEn<;b"
    "YOfw@U&xb2^W!$6nF<qFAA8Qf39U#>*Iqs{r_3_XLK0Ddu0>y6HnfH!_7YF@%WxJv%0KiO@DCb~ws8k}em8&@!jr6L^;r?N(R{UW"
    "!thW)KlPo(J=x#)!o9*^aoZ=te@ztplq?tD*-tzZZj)oEZ}>$j%a~{dKL`(KX*`riVbACy|3m&~5~JIummCc_><K@SP8KuldlD6%"
    "^H=<T%q?(v"
)
_PERM = np.frombuffer(
    zlib.decompress(base64.b85decode(_PERM_B85)), dtype="<u2"
).astype(np.int64)
_IDX = (_PERM[None, :] + _N * np.arange(_B)[:, None]).reshape(-1).astype(np.int32)

_NC = 2
_NS = 16
_NW = _NC * _NS
_PER_W = _ROWS // _NW
_CHUNK = 8
_NCHUNK = _PER_W // _CHUNK
_NBUF = 6
_NGROUP = _NCHUNK // _NBUF
_NTAIL = _NCHUNK - _NGROUP * _NBUF

@functools.cache
def _build_permute():
    mesh = plsc.VectorSubcoreMesh(core_axis_name="c", subcore_axis_name="s")

    @functools.partial(
        pl.kernel,
        mesh=mesh,
        out_type=jax.ShapeDtypeStruct((_ROWS, _D), jnp.float32),
        scratch_types=[
            pltpu.VMEM((_PER_W,), jnp.int32),
        ]
        + [pltpu.VMEM((_CHUNK, _D), jnp.float32)] * _NBUF
        + [pltpu.SemaphoreType.DMA] * (2 * _NBUF),
    )
    def _permute_rows(x_hbm, idx_hbm, out_hbm, idx_v, *scratch):
        bufs = scratch[:_NBUF]
        gsems = scratch[_NBUF : 2 * _NBUF]
        wsems = scratch[2 * _NBUF :]

        wid = lax.axis_index("s") * _NC + lax.axis_index("c")
        base = wid * _PER_W
        pltpu.sync_copy(idx_hbm.at[pl.ds(base, _PER_W)], idx_v)

        def gather_desc(k, b):
            src = x_hbm.at[idx_v.at[pl.ds(k * _CHUNK, _CHUNK)]]
            return pltpu.make_async_copy(src, bufs[b], gsems[b])

        def write_desc(k, b):
            dst = out_hbm.at[pl.ds(base + k * _CHUNK, _CHUNK)]
            return pltpu.make_async_copy(bufs[b], dst, wsems[b])

        for b in range(_NBUF):
            gather_desc(b, b).start()

        def body(g, carry):
            k0 = g * _NBUF
            for b in range(_NBUF):
                gather_desc(k0 + b, b).wait()
                write_desc(k0 + b, b).start()
            for b in range(_NBUF):
                k = k0 + b
                write_desc(k, b).wait()

                @pl.when(k + _NBUF < _NCHUNK)
                def _(b=b, k=k):
                    gather_desc(k + _NBUF, b).start()

            return carry

        lax.fori_loop(0, _NGROUP, body, 0)

        for t in range(_NTAIL):
            k = _NGROUP * _NBUF + t
            gather_desc(k, t).wait()
            write_desc(k, t).start()
        for t in range(_NTAIL):
            k = _NGROUP * _NBUF + t
            write_desc(k, t).wait()

    return _permute_rows


def kernel(x):
    xf = x.reshape(_ROWS, _D)
    out = _build_permute()(xf, jnp.asarray(_IDX))
    return out.reshape(_B, _N, _D)
